# baseline (speedup 1.0000x reference)
import functools
import math

import jax
import jax.numpy as jnp
from jax import lax
from jax.experimental import pallas as pl
from jax.experimental.pallas import tpu as pltpu
from jax.experimental.pallas import tpu_sc as plsc

F32 = jnp.float32
I32 = jnp.int32
I8 = jnp.int8
LN_EPS = 1e-5


def _dot(a, b, ca, cb):
    return lax.dot_general(a, b, ((ca, cb), ((), ())),
                           preferred_element_type=F32)


def _dotb(a, b, ca, cb):
    return lax.dot_general(a.astype(jnp.bfloat16), b.astype(jnp.bfloat16),
                           ((ca, cb), ((), ())), preferred_element_type=F32)


def _attn_ln_body(img_ref, wp_ref, bp_ref, feat_ref, gf_ref, bf_ref, gp_ref,
                  bpl_ref, w1_ref, w1p_ref, lnf_ref, lnp_ref, xw1_ref,
                  pos_scr):
    i = pl.program_id(0)

    @pl.when(i == 0)
    def _():
        pos_scr[...] = _dot(img_ref[...], wp_ref[...], (1,), (0,)) + bp_ref[...]

    pos_all = pos_scr[...]
    pos_blk = pos_scr[pl.ds(i * 128, 128), :]
    s = _dotb(pos_blk, pos_all, (1,), (1,)) * (1.0 / math.sqrt(12.0))
    m = jnp.max(s, axis=1, keepdims=True)
    p = jnp.exp(s - m)
    den = jnp.sum(p, axis=1, keepdims=True)
    attn = _dotb(p, pos_all, (1,), (0,)) / den
    mu = jnp.sum(attn, axis=1, keepdims=True) / 12.0
    lane = lax.broadcasted_iota(I32, (128, 128), 1)
    xc = jnp.where(lane < 12, attn - mu, 0.0)
    var = jnp.sum(xc * xc, axis=1, keepdims=True) / 12.0
    lnp = xc / jnp.sqrt(var + LN_EPS) * gp_ref[...] + bpl_ref[...]
    f = feat_ref[...]
    muf = jnp.sum(f, axis=1, keepdims=True) / 500.0
    lane2 = lax.broadcasted_iota(I32, (128, 512), 1)
    xcf = jnp.where(lane2 < 500, f - muf, 0.0)
    varf = jnp.sum(xcf * xcf, axis=1, keepdims=True) / 500.0
    lnf = xcf / jnp.sqrt(varf + LN_EPS) * gf_ref[...] + bf_ref[...]
    lnf_ref[...] = lnf
    lnp_ref[...] = lnp
    xw1_ref[...] = (_dot(lnf, w1_ref[...], (1,), (0,))
                    + _dot(lnp, w1p_ref[...], (1,), (0,)))


def _attn_ln(img_p, wp_p, bp_p, feat_p, gf_p, bf_p, gp_p, bpl, w1, w1p, nhid):
    n = img_p.shape[0]
    return pl.pallas_call(
        _attn_ln_body,
        grid=(n // 128,),
        in_specs=[
            pl.BlockSpec((n, 128), lambda i: (0, 0)),
            pl.BlockSpec((128, 128), lambda i: (0, 0)),
            pl.BlockSpec((1, 128), lambda i: (0, 0)),
            pl.BlockSpec((128, 512), lambda i: (i, 0)),
            pl.BlockSpec((1, 512), lambda i: (0, 0)),
            pl.BlockSpec((1, 512), lambda i: (0, 0)),
            pl.BlockSpec((1, 128), lambda i: (0, 0)),
            pl.BlockSpec((1, 128), lambda i: (0, 0)),
            pl.BlockSpec((512, nhid), lambda i: (0, 0)),
            pl.BlockSpec((128, nhid), lambda i: (0, 0)),
        ],
        out_specs=[
            pl.BlockSpec((128, 512), lambda i: (i, 0)),
            pl.BlockSpec((128, 128), lambda i: (i, 0)),
            pl.BlockSpec((128, nhid), lambda i: (i, 0)),
        ],
        out_shape=[
            jax.ShapeDtypeStruct((n, 512), F32),
            jax.ShapeDtypeStruct((n, 128), F32),
            jax.ShapeDtypeStruct((n, nhid), F32),
        ],
        scratch_shapes=[pltpu.VMEM((n, 128), F32)],
    )(img_p, wp_p, bp_p, feat_p, gf_p, bf_p, gp_p, bpl, w1, w1p)


def _dist_body(fi_ref, fj_ref, cm_ref):
    i = pl.program_id(1)
    fi = fi_ref[...]
    fj = fj_ref[...]
    sqi = jnp.sum(fi * fi, axis=1, keepdims=True)
    sqj = _dot(jnp.ones((1, 512), F32), fj * fj, (1,), (1,))
    d = sqi + sqj - 2.0 * _dotb(fi, fj, (1,), (1,))
    cm = jnp.max(d, axis=0, keepdims=True)

    @pl.when(i == 0)
    def _():
        cm_ref[...] = cm

    @pl.when(i > 0)
    def _():
        cm_ref[...] = jnp.maximum(cm_ref[...], cm)


def _dist_max(feat, bi, bj):
    n = feat.shape[0]
    return pl.pallas_call(
        _dist_body,
        grid=(n // bj, n // bi),
        in_specs=[
            pl.BlockSpec((bi, 512), lambda j, i: (i, 0)),
            pl.BlockSpec((bj, 512), lambda j, i: (j, 0)),
        ],
        out_specs=pl.BlockSpec((1, bj), lambda j, i: (0, j)),
        out_shape=jax.ShapeDtypeStruct((1, n), F32),
    )(feat, feat)


def _mask_body(has_q, fi_ref, fj_ref, t_ref, *rest):
    if has_q:
        qc_ref, qr_ref, mask_ref, deg_ref = rest
    else:
        mask_ref, deg_ref = rest
    j = pl.program_id(0)
    i = pl.program_id(1)
    bi = fi_ref.shape[0]
    bj = fj_ref.shape[0]
    fi = fi_ref[...]
    fj = fj_ref[...]
    sqi = jnp.sum(fi * fi, axis=1, keepdims=True)
    sqj = _dot(jnp.ones((1, 512), F32), fj * fj, (1,), (1,))
    d = sqi + sqj - 2.0 * _dotb(fi, fj, (1,), (1,))
    t = t_ref[0, 0]
    if has_q:
        tri = qc_ref[...] < qr_ref[...]
    else:
        gi = lax.broadcasted_iota(I32, (bi, bj), 0) + i * bi
        gj = lax.broadcasted_iota(I32, (bi, bj), 1) + j * bj
        tri = gi < gj
    m = jnp.where((d < t) & tri, 1.0, 0.0)
    mask_ref[...] = m.astype(I8)
    s = jnp.sum(m, axis=0, keepdims=True)

    @pl.when(i == 0)
    def _():
        deg_ref[...] = 1.0 + s

    @pl.when(i > 0)
    def _():
        deg_ref[...] += s


def _mask_deg(featsub, qc, qr, t, bi, bj):
    n = featsub.shape[0]
    has_q = qc is not None
    in_specs = [
        pl.BlockSpec((bi, 512), lambda j, i: (i, 0)),
        pl.BlockSpec((bj, 512), lambda j, i: (j, 0)),
        pl.BlockSpec(memory_space=pltpu.SMEM),
    ]
    args = [featsub, featsub, t]
    if has_q:
        in_specs += [
            pl.BlockSpec((bi, 1), lambda j, i: (i, 0)),
            pl.BlockSpec((1, bj), lambda j, i: (0, j)),
        ]
        args += [qc, qr]
    return pl.pallas_call(
        functools.partial(_mask_body, has_q),
        grid=(n // bj, n // bi),
        in_specs=in_specs,
        out_specs=[
            pl.BlockSpec((bi, bj), lambda j, i: (i, j)),
            pl.BlockSpec((1, bj), lambda j, i: (0, j)),
        ],
        out_shape=[
            jax.ShapeDtypeStruct((n, n), I8),
            jax.ShapeDtypeStruct((1, n), F32),
        ],
    )(*args)


def _conv_body(fused, mask_ref, x_ref, degi_ref, degj_ref, b_ref, *rest):
    if fused:
        v_ref, w_ref, out_ref, mx_ref, sm_ref = rest
    else:
        out_ref, = rest
    j = pl.program_id(0)
    i = pl.program_id(1)
    ni = pl.num_programs(1)
    bi = mask_ref.shape[0]
    bj = mask_ref.shape[1]
    dinv_i = 1.0 / jnp.sqrt(degi_ref[...])
    if fused:
        xs = x_ref[...] * jnp.tanh(v_ref[...])
        xw = _dot(xs, w_ref[...], (1,), (0,))

        @pl.when(j == 0)
        def _():
            mro = jnp.max(xs, axis=0, keepdims=True)
            sro = jnp.sum(xs, axis=0, keepdims=True)

            @pl.when(i == 0)
            def _():
                mx_ref[...] = mro
                sm_ref[...] = sro

            @pl.when(i > 0)
            def _():
                mx_ref[...] = jnp.maximum(mx_ref[...], mro)
                sm_ref[...] += sro
    else:
        xw = x_ref[...]
    z = xw * dinv_i
    m = mask_ref[...].astype(jnp.bfloat16)
    contrib = _dotb(m, z, (0,), (0,))

    @pl.when(i == 0)
    def _():
        out_ref[...] = contrib

    @pl.when(i > 0)
    def _():
        out_ref[...] += contrib

    off = i * bi - j * bj

    @pl.when((off >= 0) & (off < bj))
    def _():
        out_ref[pl.ds(pl.multiple_of(off, bi), bi), :] += z

    @pl.when(i == ni - 1)
    def _():
        dinv_j = 1.0 / jnp.sqrt(degj_ref[...])
        out_ref[...] = jnp.maximum(out_ref[...] * dinv_j + b_ref[...], 0.0)


def _conv(mask8, xin, degc, b_row, vals_col, w_next, bi, bj):
    n = mask8.shape[0]
    nhid = xin.shape[1]
    fused = vals_col is not None
    in_specs = [
        pl.BlockSpec((bi, bj), lambda j, i: (i, j)),
        pl.BlockSpec((bi, nhid), lambda j, i: (i, 0)),
        pl.BlockSpec((bi, 1), lambda j, i: (i, 0)),
        pl.BlockSpec((bj, 1), lambda j, i: (j, 0)),
        pl.BlockSpec((1, nhid), lambda j, i: (0, 0)),
    ]
    args = [mask8, xin, degc, degc, b_row]
    out_specs = [pl.BlockSpec((bj, nhid), lambda j, i: (j, 0))]
    out_shape = [jax.ShapeDtypeStruct((n, nhid), F32)]
    if fused:
        in_specs += [
            pl.BlockSpec((bi, 1), lambda j, i: (i, 0)),
            pl.BlockSpec((nhid, nhid), lambda j, i: (0, 0)),
        ]
        args += [vals_col, w_next]
        out_specs += [
            pl.BlockSpec((1, nhid), lambda j, i: (0, 0)),
            pl.BlockSpec((1, nhid), lambda j, i: (0, 0)),
        ]
        out_shape += [
            jax.ShapeDtypeStruct((1, nhid), F32),
            jax.ShapeDtypeStruct((1, nhid), F32),
        ]
    res = pl.pallas_call(
        functools.partial(_conv_body, fused),
        grid=(n // bj, n // bi),
        in_specs=in_specs,
        out_specs=out_specs,
        out_shape=out_shape,
    )(*args)
    return res if fused else res[0]


def _score_body(mask_ref, x_ref, degi_ref, degj_ref, ws_ref, bs_ref, out_ref):
    j = pl.program_id(0)
    i = pl.program_id(1)
    ni = pl.num_programs(1)
    bi = mask_ref.shape[0]
    bj = mask_ref.shape[1]
    dinv_i = 1.0 / jnp.sqrt(degi_ref[...])
    u = _dot(x_ref[...], ws_ref[...], (1,), (0,)) * dinv_i
    m = mask_ref[...].astype(jnp.bfloat16)
    contrib = _dotb(m, u, (0,), (0,))

    @pl.when(i == 0)
    def _():
        out_ref[...] = contrib

    @pl.when(i > 0)
    def _():
        out_ref[...] += contrib

    off = i * bi - j * bj

    @pl.when((off >= 0) & (off < bj))
    def _():
        out_ref[pl.ds(pl.multiple_of(off, bi), bi), :] += u

    @pl.when(i == ni - 1)
    def _():
        dinv_j = 1.0 / jnp.sqrt(degj_ref[...])
        out_ref[...] = out_ref[...] * dinv_j + bs_ref[...]


def _score(mask8, x, degc, ws_p, bs_row, bi, bj):
    n = mask8.shape[0]
    nhid = x.shape[1]
    out = pl.pallas_call(
        _score_body,
        grid=(n // bj, n // bi),
        in_specs=[
            pl.BlockSpec((bi, bj), lambda j, i: (i, j)),
            pl.BlockSpec((bi, nhid), lambda j, i: (i, 0)),
            pl.BlockSpec((bi, 1), lambda j, i: (i, 0)),
            pl.BlockSpec((bj, 1), lambda j, i: (j, 0)),
            pl.BlockSpec((nhid, 128), lambda j, i: (0, 0)),
            pl.BlockSpec((1, 128), lambda j, i: (0, 0)),
        ],
        out_specs=pl.BlockSpec((bj, 128), lambda j, i: (j, 0)),
        out_shape=jax.ShapeDtypeStruct((n, 128), F32),
    )(mask8, x, degc, degc, ws_p, bs_row)
    return out[:, :1]


def _rank_body(sc_ref, sr_ref, rank_ref):
    i = pl.program_id(0)
    bi = sc_ref.shape[0]
    n = sr_ref.shape[1]
    si = sc_ref[...]
    sj = sr_ref[...]
    gi = lax.broadcasted_iota(I32, (bi, n), 0) + i * bi
    gj = lax.broadcasted_iota(I32, (bi, n), 1)
    before = (sj > si) | ((sj == si) & (gj < gi))
    rank_ref[...] = jnp.sum(before.astype(I32), axis=1, keepdims=True)


def _rank(sc, sr):
    n = sc.shape[0]
    bi = 256
    return pl.pallas_call(
        _rank_body,
        grid=(n // bi,),
        in_specs=[
            pl.BlockSpec((bi, 1), lambda i: (i, 0)),
            pl.BlockSpec((1, n), lambda i: (0, 0)),
        ],
        out_specs=pl.BlockSpec((bi, 1), lambda i: (i, 0)),
        out_shape=jax.ShapeDtypeStruct((n, 1), I32),
    )(sc, sr)


def _topk_select(rank_flat, s_flat, q_flat, k_out):
    k_in = rank_flat.shape[0]
    with_q = q_flat is not None
    mesh = plsc.VectorSubcoreMesh(core_axis_name="c", subcore_axis_name="s")
    out_type = [jax.ShapeDtypeStruct((k_out,), I32),
                jax.ShapeDtypeStruct((k_out,), F32)]
    scratch = [pltpu.VMEM((k_in,), I32), pltpu.VMEM((k_in,), F32),
               pltpu.VMEM((k_out,), I32), pltpu.VMEM((k_out,), F32)]
    if with_q:
        out_type.append(jax.ShapeDtypeStruct((k_out,), I32))
        scratch += [pltpu.VMEM((k_in,), I32), pltpu.VMEM((k_out,), I32)]

    def body(*refs):
        if with_q:
            (rank_hbm, s_hbm, q_hbm, perm_out, vals_out, q_out,
             rank_v, s_v, perm_v, vals_v, q_v, qn_v) = refs
        else:
            (rank_hbm, s_hbm, perm_out, vals_out,
             rank_v, s_v, perm_v, vals_v) = refs
        cid = lax.axis_index("c")
        sid = lax.axis_index("s")

        @pl.when(jnp.logical_and(cid == 0, sid == 0))
        def _():
            pltpu.sync_copy(rank_hbm, rank_v)
            pltpu.sync_copy(s_hbm, s_v)
            if with_q:
                pltpu.sync_copy(q_hbm, q_v)

            def step(c, carry):
                base = c * 16
                idx = rank_v[pl.ds(base, 16)]
                msk = idx < k_out
                ids = lax.iota(I32, 16) + base
                plsc.store_scatter(perm_v, [idx], ids, mask=msk)
                plsc.store_scatter(vals_v, [idx], s_v[pl.ds(base, 16)],
                                   mask=msk)
                if with_q:
                    plsc.store_scatter(qn_v, [idx], q_v[pl.ds(base, 16)],
                                       mask=msk)
                return carry

            lax.fori_loop(0, k_in // 16, step, 0)
            pltpu.sync_copy(perm_v, perm_out)
            pltpu.sync_copy(vals_v, vals_out)
            if with_q:
                pltpu.sync_copy(qn_v, q_out)

    fn = pl.kernel(body, out_type=tuple(out_type), mesh=mesh,
                   scratch_types=tuple(scratch),
                   compiler_params=pltpu.CompilerParams(
                       needs_layout_passes=False))
    if with_q:
        return fn(rank_flat, s_flat, q_flat)
    return fn(rank_flat, s_flat)


def _gather_rows(idx, tables, n_out):
    n_workers = 32
    c = n_out // n_workers
    if c % 8 != 0:
        c = 64
        n_workers = n_out // c
    assert c * n_workers == n_out and c % 8 == 0 and c <= 128
    nt = len(tables)
    mesh = plsc.VectorSubcoreMesh(core_axis_name="c", subcore_axis_name="s")
    out_type = tuple(jax.ShapeDtypeStruct((n_out, tb.shape[1]), F32)
                     for tb in tables)
    scratch = tuple([pltpu.VMEM((c,), I32)]
                    + [pltpu.VMEM((c, tb.shape[1]), F32) for tb in tables]
                    + [pltpu.SemaphoreType.DMA])

    def body(*refs):
        idx_hbm = refs[0]
        tabs = refs[1:1 + nt]
        outs = refs[1 + nt:1 + 2 * nt]
        idx_v = refs[1 + 2 * nt]
        bufs = refs[2 + 2 * nt:2 + 3 * nt]
        sem = refs[2 + 3 * nt]
        cid = lax.axis_index("c")
        sid = lax.axis_index("s")
        wid = sid * 2 + cid

        @pl.when(wid < n_workers)
        def _():
            base = wid * c
            pltpu.sync_copy(idx_hbm.at[pl.ds(base, c)], idx_v)
            for tb, buf, out in zip(tabs, bufs, outs):
                pltpu.async_copy(tb.at[idx_v], buf, sem).wait()
                pltpu.sync_copy(buf, out.at[pl.ds(base, c)])

    fn = pl.kernel(body, out_type=out_type, mesh=mesh, scratch_types=scratch,
                   compiler_params=pltpu.CompilerParams(
                       needs_layout_passes=False))
    res = fn(idx, *tables)
    if not isinstance(res, (list, tuple)):
        res = (res,)
    return list(res)


def _readout_body(x_ref, v_ref, mx_ref, sm_ref):
    i = pl.program_id(0)
    xs = x_ref[...] * jnp.tanh(v_ref[...])
    m = jnp.max(xs, axis=0, keepdims=True)
    s = jnp.sum(xs, axis=0, keepdims=True)

    @pl.when(i == 0)
    def _():
        mx_ref[...] = m
        sm_ref[...] = s

    @pl.when(i > 0)
    def _():
        mx_ref[...] = jnp.maximum(mx_ref[...], m)
        sm_ref[...] += s


def _readout(x_gath, vals_col, br):
    n, nhid = x_gath.shape
    return pl.pallas_call(
        _readout_body,
        grid=(n // br,),
        in_specs=[
            pl.BlockSpec((br, nhid), lambda i: (i, 0)),
            pl.BlockSpec((br, 1), lambda i: (i, 0)),
        ],
        out_specs=[
            pl.BlockSpec((1, nhid), lambda i: (0, 0)),
            pl.BlockSpec((1, nhid), lambda i: (0, 0)),
        ],
        out_shape=[
            jax.ShapeDtypeStruct((1, nhid), F32),
            jax.ShapeDtypeStruct((1, nhid), F32),
        ],
    )(x_gath, vals_col)


def kernel(feature, img_info, W_pos, b_pos, g_f, b_f, g_p, b_p,
           W1, b1, W2, b2, W3, b3, Ws1, bs1, Ws2, bs2, Ws3, bs3):
    n = feature.shape[0]
    nf = feature.shape[1]
    nhid = W1.shape[1]
    k1 = math.ceil(0.75 * n)
    k2 = math.ceil(0.75 * k1)
    k3 = math.ceil(0.75 * k2)

    img_p = jnp.pad(img_info, ((0, 0), (0, 128 - img_info.shape[1])))
    wp_p = jnp.pad(W_pos, ((0, 128 - W_pos.shape[0]), (0, 128 - W_pos.shape[1])))
    bp_p = jnp.pad(b_pos, (0, 128 - b_pos.shape[0])).reshape(1, 128)
    feat_p = jnp.pad(feature, ((0, 0), (0, 512 - nf)))
    gf_p = jnp.pad(g_f, (0, 512 - nf)).reshape(1, 512)
    bf_p = jnp.pad(b_f, (0, 512 - nf)).reshape(1, 512)
    gp_p = jnp.pad(g_p, (0, 128 - g_p.shape[0])).reshape(1, 128)
    bpl = jnp.pad(b_p, (0, 128 - b_p.shape[0])).reshape(1, 128)
    w1p = jnp.pad(W1[nf:, :], ((0, 128 - (512 - nf)), (0, 0)))
    ws1_p = jnp.pad(Ws1, ((0, 0), (0, 127)))
    ws2_p = jnp.pad(Ws2, ((0, 0), (0, 127)))
    ws3_p = jnp.pad(Ws3, ((0, 0), (0, 127)))
    bs1_r = jnp.broadcast_to(bs1.reshape(1, 1), (1, 128))
    bs2_r = jnp.broadcast_to(bs2.reshape(1, 1), (1, 128))
    bs3_r = jnp.broadcast_to(bs3.reshape(1, 1), (1, 128))
    b1_r = b1.reshape(1, nhid)
    b2_r = b2.reshape(1, nhid)
    b3_r = b3.reshape(1, nhid)

    lnf, lnp, xw1 = _attn_ln(img_p, wp_p, bp_p, feat_p, gf_p, bf_p, gp_p,
                             bpl, W1, w1p, nhid)
    feat = jnp.concatenate([lnf[:, :nf], lnp[:, :512 - nf]], axis=1)

    colmax = _dist_max(feat, 256, 1024)
    t = (0.5 * jnp.max(colmax)).reshape(1, 1)

    mask1, deg1 = _mask_deg(feat, None, None, t, 256, 1024)
    deg1c = deg1.reshape(n, 1)
    x1 = _conv(mask1, xw1, deg1c, b1_r, None, None, 256, 1024)
    s1 = _score(mask1, x1, deg1c, ws1_p, bs1_r, 256, 1024)
    rank1 = _rank(s1, s1.reshape(1, n))
    perm1, vals1 = _topk_select(rank1.reshape(n), s1.reshape(n), None, k1)
    featsub2, xg1 = _gather_rows(perm1, [feat, x1], k1)

    q2c = perm1.reshape(k1, 1)
    q2r = perm1.reshape(1, k1)
    mask2, deg2 = _mask_deg(featsub2, q2c, q2r, t, 256, 1024)
    deg2c = deg2.reshape(k1, 1)
    x2, mx1, sm1 = _conv(mask2, xg1, deg2c, b2_r, vals1.reshape(k1, 1), W2,
                         256, 1024)
    read1 = jnp.concatenate([mx1, sm1 / k1], axis=1)
    s2 = _score(mask2, x2, deg2c, ws2_p, bs2_r, 256, 1024)
    rank2 = _rank(s2, s2.reshape(1, k1))
    perm2, vals2, q3 = _topk_select(rank2.reshape(k1), s2.reshape(k1),
                                    perm1, k2)
    featsub3, xg2 = _gather_rows(perm2, [featsub2, x2], k2)

    q3c = q3.reshape(k2, 1)
    q3r = q3.reshape(1, k2)
    mask3, deg3 = _mask_deg(featsub3, q3c, q3r, t, 256, 768)
    deg3c = deg3.reshape(k2, 1)
    x3, mx2, sm2 = _conv(mask3, xg2, deg3c, b3_r, vals2.reshape(k2, 1), W3,
                         256, 768)
    read2 = jnp.concatenate([mx2, sm2 / k2], axis=1)
    s3 = _score(mask3, x3, deg3c, ws3_p, bs3_r, 256, 768)
    rank3 = _rank(s3, s3.reshape(1, k2))
    perm3, vals3 = _topk_select(rank3.reshape(k2), s3.reshape(k2), None, k3)
    (xg3,) = _gather_rows(perm3, [x3], k3)
    mx3, sm3 = _readout(xg3, vals3.reshape(k3, 1), 64)
    read3 = jnp.concatenate([mx3, sm3 / k3], axis=1)

    return read1 + read2 + read3


# speedup vs baseline: 1.4154x; 1.4154x over previous
import functools
import math

import jax
import jax.numpy as jnp
from jax import lax
from jax.experimental import pallas as pl
from jax.experimental.pallas import tpu as pltpu
from jax.experimental.pallas import tpu_sc as plsc

F32 = jnp.float32
BF16 = jnp.bfloat16
I32 = jnp.int32
LN_EPS = 1e-5


def _dot(a, b, ca, cb):
    return lax.dot_general(a, b, ((ca, cb), ((), ())),
                           preferred_element_type=F32)


def _dotb(a, b, ca, cb):
    return lax.dot_general(a.astype(BF16), b.astype(BF16),
                           ((ca, cb), ((), ())), preferred_element_type=F32)


def _attn_ln_body(img_ref, wp_ref, bp_ref, feat_ref, gf_ref, bf_ref, gp_ref,
                  bpl_ref, w1_ref, w1p_ref, lnf_ref, lnp_ref, xw1_ref,
                  pos_scr):
    i = pl.program_id(0)

    @pl.when(i == 0)
    def _():
        pos_scr[...] = _dot(img_ref[...], wp_ref[...], (1,), (0,)) + bp_ref[...]

    pos_all = pos_scr[...]
    pos_blk = pos_scr[pl.ds(i * 128, 128), :]
    s = _dotb(pos_blk, pos_all, (1,), (1,)) * (1.0 / math.sqrt(12.0))
    m = jnp.max(s, axis=1, keepdims=True)
    p = jnp.exp(s - m)
    den = jnp.sum(p, axis=1, keepdims=True)
    attn = _dotb(p, pos_all, (1,), (0,)) / den
    mu = jnp.sum(attn, axis=1, keepdims=True) / 12.0
    lane = lax.broadcasted_iota(I32, (128, 128), 1)
    xc = jnp.where(lane < 12, attn - mu, 0.0)
    var = jnp.sum(xc * xc, axis=1, keepdims=True) / 12.0
    lnp = xc / jnp.sqrt(var + LN_EPS) * gp_ref[...] + bpl_ref[...]
    f = feat_ref[...]
    muf = jnp.sum(f, axis=1, keepdims=True) / 500.0
    lane2 = lax.broadcasted_iota(I32, (128, 512), 1)
    xcf = jnp.where(lane2 < 500, f - muf, 0.0)
    varf = jnp.sum(xcf * xcf, axis=1, keepdims=True) / 500.0
    lnf = xcf / jnp.sqrt(varf + LN_EPS) * gf_ref[...] + bf_ref[...]
    lnf_ref[...] = lnf
    lnp_ref[...] = lnp
    xw1_ref[...] = (_dot(lnf, w1_ref[...], (1,), (0,))
                    + _dot(lnp, w1p_ref[...], (1,), (0,)))


def _attn_ln(img_p, wp_p, bp_p, feat_p, gf_p, bf_p, gp_p, bpl, w1, w1p, nhid):
    n = img_p.shape[0]
    return pl.pallas_call(
        _attn_ln_body,
        grid=(n // 128,),
        in_specs=[
            pl.BlockSpec((n, 128), lambda i: (0, 0)),
            pl.BlockSpec((128, 128), lambda i: (0, 0)),
            pl.BlockSpec((1, 128), lambda i: (0, 0)),
            pl.BlockSpec((128, 512), lambda i: (i, 0)),
            pl.BlockSpec((1, 512), lambda i: (0, 0)),
            pl.BlockSpec((1, 512), lambda i: (0, 0)),
            pl.BlockSpec((1, 128), lambda i: (0, 0)),
            pl.BlockSpec((1, 128), lambda i: (0, 0)),
            pl.BlockSpec((512, nhid), lambda i: (0, 0)),
            pl.BlockSpec((128, nhid), lambda i: (0, 0)),
        ],
        out_specs=[
            pl.BlockSpec((128, 512), lambda i: (i, 0)),
            pl.BlockSpec((128, 128), lambda i: (i, 0)),
            pl.BlockSpec((128, nhid), lambda i: (i, 0)),
        ],
        out_shape=[
            jax.ShapeDtypeStruct((n, 512), F32),
            jax.ShapeDtypeStruct((n, 128), F32),
            jax.ShapeDtypeStruct((n, nhid), F32),
        ],
        scratch_shapes=[pltpu.VMEM((n, 128), F32)],
    )(img_p, wp_p, bp_p, feat_p, gf_p, bf_p, gp_p, bpl, w1, w1p)


def _dist_body(fi_ref, fj_ref, d_ref, cm_ref):
    i = pl.program_id(1)
    fi = fi_ref[...]
    fj = fj_ref[...]
    sqi = jnp.sum(fi * fi, axis=1, keepdims=True)
    sqj = _dot(jnp.ones((1, 512), F32), fj * fj, (1,), (1,))
    d = sqi + sqj - 2.0 * _dotb(fi, fj, (1,), (1,))
    d_ref[...] = d.astype(BF16)
    cm = jnp.max(d, axis=0, keepdims=True)

    @pl.when(i == 0)
    def _():
        cm_ref[...] = cm

    @pl.when(i > 0)
    def _():
        cm_ref[...] = jnp.maximum(cm_ref[...], cm)


def _dist_max(feat, bi, bj):
    n = feat.shape[0]
    return pl.pallas_call(
        _dist_body,
        grid=(n // bj, n // bi),
        in_specs=[
            pl.BlockSpec((bi, 512), lambda j, i: (i, 0)),
            pl.BlockSpec((bj, 512), lambda j, i: (j, 0)),
        ],
        out_specs=[
            pl.BlockSpec((bi, bj), lambda j, i: (i, j)),
            pl.BlockSpec((1, bj), lambda j, i: (0, j)),
        ],
        out_shape=[
            jax.ShapeDtypeStruct((n, n), BF16),
            jax.ShapeDtypeStruct((1, n), F32),
        ],
    )(feat, feat)


def _mask1_body(d_ref, t_ref, mask_ref, dinv_ref):
    j = pl.program_id(1)
    i = pl.program_id(0)
    nj = pl.num_programs(1)
    bi, bj = d_ref.shape
    t = t_ref[0, 0]
    gi = lax.broadcasted_iota(I32, (bi, bj), 0) + i * bi
    gj = lax.broadcasted_iota(I32, (bi, bj), 1) + j * bj
    cond = (d_ref[...].astype(F32) < t) & (gj < gi)
    m = jnp.where(cond, 1.0, 0.0).astype(BF16)
    mask_ref[...] = m
    degp = _dot(m, jnp.ones((bj, 128), BF16), (1,), (0,))[:, :1]

    @pl.when(j == 0)
    def _():
        dinv_ref[...] = 1.0 + degp

    @pl.when(j > 0)
    def _():
        dinv_ref[...] += degp

    @pl.when(j == nj - 1)
    def _():
        dinv_ref[...] = 1.0 / jnp.sqrt(dinv_ref[...])


def _mask1_deg(d_bf, t, bi, bj):
    n = d_bf.shape[0]
    return pl.pallas_call(
        _mask1_body,
        grid=(n // bi, n // bj),
        in_specs=[
            pl.BlockSpec((bi, bj), lambda i, j: (i, j)),
            pl.BlockSpec(memory_space=pltpu.SMEM),
        ],
        out_specs=[
            pl.BlockSpec((bi, bj), lambda i, j: (i, j)),
            pl.BlockSpec((bi, 1), lambda i, j: (i, 0)),
        ],
        out_shape=[
            jax.ShapeDtypeStruct((n, n), BF16),
            jax.ShapeDtypeStruct((n, 1), F32),
        ],
    )(d_bf, t)


def _mask23_body(fi_ref, fj_ref, qc_ref, qr_ref, t_ref, mask_ref, dinv_ref):
    j = pl.program_id(1)
    nj = pl.num_programs(1)
    bi = fi_ref.shape[0]
    bj = fj_ref.shape[0]
    fi = fi_ref[...]
    fj = fj_ref[...]
    sqi = jnp.sum(fi * fi, axis=1, keepdims=True)
    sqj = _dot(jnp.ones((1, 512), F32), fj * fj, (1,), (1,))
    d = sqi + sqj - 2.0 * _dotb(fi, fj, (1,), (1,))
    t = t_ref[0, 0]
    cond = (d < t) & (qr_ref[...] < qc_ref[...])
    m = jnp.where(cond, 1.0, 0.0).astype(BF16)
    mask_ref[...] = m
    degp = _dot(m, jnp.ones((bj, 128), BF16), (1,), (0,))[:, :1]

    @pl.when(j == 0)
    def _():
        dinv_ref[...] = 1.0 + degp

    @pl.when(j > 0)
    def _():
        dinv_ref[...] += degp

    @pl.when(j == nj - 1)
    def _():
        dinv_ref[...] = 1.0 / jnp.sqrt(dinv_ref[...])


def _mask23_deg(featsub, qc, qr, t, bi, bj):
    n = featsub.shape[0]
    return pl.pallas_call(
        _mask23_body,
        grid=(n // bi, n // bj),
        in_specs=[
            pl.BlockSpec((bi, 512), lambda i, j: (i, 0)),
            pl.BlockSpec((bj, 512), lambda i, j: (j, 0)),
            pl.BlockSpec((bi, 1), lambda i, j: (i, 0)),
            pl.BlockSpec((1, bj), lambda i, j: (0, j)),
            pl.BlockSpec(memory_space=pltpu.SMEM),
        ],
        out_specs=[
            pl.BlockSpec((bi, bj), lambda i, j: (i, j)),
            pl.BlockSpec((bi, 1), lambda i, j: (i, 0)),
        ],
        out_shape=[
            jax.ShapeDtypeStruct((n, n), BF16),
            jax.ShapeDtypeStruct((n, 1), F32),
        ],
    )(featsub, featsub, qc, qr, t)


def _conv_body(fused, mask_ref, x_ref, dinvj_ref, dinvi_ref, b_ref, *rest):
    if fused:
        v_ref, w_ref, out_ref, mx_ref, sm_ref = rest
    else:
        out_ref, = rest
    i = pl.program_id(0)
    j = pl.program_id(1)
    nj = pl.num_programs(1)
    bi = mask_ref.shape[0]
    bj = mask_ref.shape[1]
    if fused:
        xs = x_ref[...] * jnp.tanh(v_ref[...])
        xw = _dot(xs, w_ref[...], (1,), (0,))

        @pl.when(i == 0)
        def _():
            mro = jnp.max(xs, axis=0, keepdims=True)
            sro = jnp.sum(xs, axis=0, keepdims=True)

            @pl.when(j == 0)
            def _():
                mx_ref[...] = mro
                sm_ref[...] = sro

            @pl.when(j > 0)
            def _():
                mx_ref[...] = jnp.maximum(mx_ref[...], mro)
                sm_ref[...] += sro
    else:
        xw = x_ref[...]
    z = xw * dinvj_ref[...]
    contrib = _dotb(mask_ref[...], z, (1,), (0,))

    @pl.when(j == 0)
    def _():
        out_ref[...] = contrib

    @pl.when(j > 0)
    def _():
        out_ref[...] += contrib

    off = j * bj - i * bi

    @pl.when((off >= 0) & (off < bi))
    def _():
        out_ref[pl.ds(pl.multiple_of(off, bj), bj), :] += z

    @pl.when(j == nj - 1)
    def _():
        out_ref[...] = jnp.maximum(out_ref[...] * dinvi_ref[...] + b_ref[...],
                                   0.0)


def _conv(maskT, xin, dinv, b_row, vals_col, w_next, bi, bj):
    n = maskT.shape[0]
    nhid = xin.shape[1]
    fused = vals_col is not None
    in_specs = [
        pl.BlockSpec((bi, bj), lambda i, j: (i, j)),
        pl.BlockSpec((bj, nhid), lambda i, j: (j, 0)),
        pl.BlockSpec((bj, 1), lambda i, j: (j, 0)),
        pl.BlockSpec((bi, 1), lambda i, j: (i, 0)),
        pl.BlockSpec((1, nhid), lambda i, j: (0, 0)),
    ]
    args = [maskT, xin, dinv, dinv, b_row]
    out_specs = [pl.BlockSpec((bi, nhid), lambda i, j: (i, 0))]
    out_shape = [jax.ShapeDtypeStruct((n, nhid), F32)]
    if fused:
        in_specs += [
            pl.BlockSpec((bj, 1), lambda i, j: (j, 0)),
            pl.BlockSpec((nhid, nhid), lambda i, j: (0, 0)),
        ]
        args += [vals_col, w_next]
        out_specs += [
            pl.BlockSpec((1, nhid), lambda i, j: (0, 0)),
            pl.BlockSpec((1, nhid), lambda i, j: (0, 0)),
        ]
        out_shape += [
            jax.ShapeDtypeStruct((1, nhid), F32),
            jax.ShapeDtypeStruct((1, nhid), F32),
        ]
    res = pl.pallas_call(
        functools.partial(_conv_body, fused),
        grid=(n // bi, n // bj),
        in_specs=in_specs,
        out_specs=out_specs,
        out_shape=out_shape,
    )(*args)
    return res if fused else res[0]


def _score_body(mask_ref, x_ref, dinvj_ref, dinvi_ref, ws_ref, bs_ref,
                out_ref):
    i = pl.program_id(0)
    j = pl.program_id(1)
    nj = pl.num_programs(1)
    bi = mask_ref.shape[0]
    bj = mask_ref.shape[1]
    u = _dot(x_ref[...], ws_ref[...], (1,), (0,)) * dinvj_ref[...]
    contrib = _dotb(mask_ref[...], u, (1,), (0,))

    @pl.when(j == 0)
    def _():
        out_ref[...] = contrib

    @pl.when(j > 0)
    def _():
        out_ref[...] += contrib

    off = j * bj - i * bi

    @pl.when((off >= 0) & (off < bi))
    def _():
        out_ref[pl.ds(pl.multiple_of(off, bj), bj), :] += u

    @pl.when(j == nj - 1)
    def _():
        out_ref[...] = out_ref[...] * dinvi_ref[...] + bs_ref[...]


def _score(maskT, x, dinv, ws_p, bs_row, bi, bj):
    n = maskT.shape[0]
    nhid = x.shape[1]
    out = pl.pallas_call(
        _score_body,
        grid=(n // bi, n // bj),
        in_specs=[
            pl.BlockSpec((bi, bj), lambda i, j: (i, j)),
            pl.BlockSpec((bj, nhid), lambda i, j: (j, 0)),
            pl.BlockSpec((bj, 1), lambda i, j: (j, 0)),
            pl.BlockSpec((bi, 1), lambda i, j: (i, 0)),
            pl.BlockSpec((nhid, 128), lambda i, j: (0, 0)),
            pl.BlockSpec((1, 128), lambda i, j: (0, 0)),
        ],
        out_specs=pl.BlockSpec((bi, 128), lambda i, j: (i, 0)),
        out_shape=jax.ShapeDtypeStruct((n, 128), F32),
    )(maskT, x, dinv, dinv, ws_p, bs_row)
    return out[:, :1]


def _rank_body(sc_ref, sr_ref, rank_ref):
    i = pl.program_id(0)
    bi = sc_ref.shape[0]
    n = sr_ref.shape[1]
    si = sc_ref[...]
    sj = sr_ref[...]
    gi = lax.broadcasted_iota(I32, (bi, n), 0) + i * bi
    gj = lax.broadcasted_iota(I32, (bi, n), 1)
    before = (sj > si) | ((sj == si) & (gj < gi))
    rank_ref[...] = jnp.sum(before.astype(I32), axis=1, keepdims=True)


def _rank(sc, sr):
    n = sc.shape[0]
    bi = 256
    return pl.pallas_call(
        _rank_body,
        grid=(n // bi,),
        in_specs=[
            pl.BlockSpec((bi, 1), lambda i: (i, 0)),
            pl.BlockSpec((1, n), lambda i: (0, 0)),
        ],
        out_specs=pl.BlockSpec((bi, 1), lambda i: (i, 0)),
        out_shape=jax.ShapeDtypeStruct((n, 1), I32),
    )(sc, sr)


def _topk_select(rank_flat, s_flat, q_flat, k_out):
    k_in = rank_flat.shape[0]
    with_q = q_flat is not None
    mesh = plsc.VectorSubcoreMesh(core_axis_name="c", subcore_axis_name="s")
    out_type = [jax.ShapeDtypeStruct((k_out,), I32),
                jax.ShapeDtypeStruct((k_out,), F32)]
    scratch = [pltpu.VMEM((k_in,), I32), pltpu.VMEM((k_in,), F32),
               pltpu.VMEM((k_out,), I32), pltpu.VMEM((k_out,), F32)]
    if with_q:
        out_type.append(jax.ShapeDtypeStruct((k_out,), I32))
        scratch += [pltpu.VMEM((k_in,), I32), pltpu.VMEM((k_out,), I32)]

    def body(*refs):
        if with_q:
            (rank_hbm, s_hbm, q_hbm, perm_out, vals_out, q_out,
             rank_v, s_v, perm_v, vals_v, q_v, qn_v) = refs
        else:
            (rank_hbm, s_hbm, perm_out, vals_out,
             rank_v, s_v, perm_v, vals_v) = refs
        cid = lax.axis_index("c")
        sid = lax.axis_index("s")

        @pl.when(jnp.logical_and(cid == 0, sid == 0))
        def _():
            pltpu.sync_copy(rank_hbm, rank_v)
            pltpu.sync_copy(s_hbm, s_v)
            if with_q:
                pltpu.sync_copy(q_hbm, q_v)

            def step(c, carry):
                base = c * 16
                idx = rank_v[pl.ds(base, 16)]
                msk = idx < k_out
                ids = lax.iota(I32, 16) + base
                plsc.store_scatter(perm_v, [idx], ids, mask=msk)
                plsc.store_scatter(vals_v, [idx], s_v[pl.ds(base, 16)],
                                   mask=msk)
                if with_q:
                    plsc.store_scatter(qn_v, [idx], q_v[pl.ds(base, 16)],
                                       mask=msk)
                return carry

            lax.fori_loop(0, k_in // 16, step, 0)
            pltpu.sync_copy(perm_v, perm_out)
            pltpu.sync_copy(vals_v, vals_out)
            if with_q:
                pltpu.sync_copy(qn_v, q_out)

    fn = pl.kernel(body, out_type=tuple(out_type), mesh=mesh,
                   scratch_types=tuple(scratch),
                   compiler_params=pltpu.CompilerParams(
                       needs_layout_passes=False))
    if with_q:
        return fn(rank_flat, s_flat, q_flat)
    return fn(rank_flat, s_flat)


def _gather_rows(idx, tables, n_out):
    n_workers = 32
    c = n_out // n_workers
    if c % 8 != 0:
        c = 64
        n_workers = n_out // c
    assert c * n_workers == n_out and c % 8 == 0 and c <= 128
    nt = len(tables)
    mesh = plsc.VectorSubcoreMesh(core_axis_name="c", subcore_axis_name="s")
    out_type = tuple(jax.ShapeDtypeStruct((n_out, tb.shape[1]), F32)
                     for tb in tables)
    scratch = tuple([pltpu.VMEM((c,), I32)]
                    + [pltpu.VMEM((c, tb.shape[1]), F32) for tb in tables]
                    + [pltpu.SemaphoreType.DMA])

    def body(*refs):
        idx_hbm = refs[0]
        tabs = refs[1:1 + nt]
        outs = refs[1 + nt:1 + 2 * nt]
        idx_v = refs[1 + 2 * nt]
        bufs = refs[2 + 2 * nt:2 + 3 * nt]
        sem = refs[2 + 3 * nt]
        cid = lax.axis_index("c")
        sid = lax.axis_index("s")
        wid = sid * 2 + cid

        @pl.when(wid < n_workers)
        def _():
            base = wid * c
            pltpu.sync_copy(idx_hbm.at[pl.ds(base, c)], idx_v)
            for tb, buf, out in zip(tabs, bufs, outs):
                pltpu.async_copy(tb.at[idx_v], buf, sem).wait()
                pltpu.sync_copy(buf, out.at[pl.ds(base, c)])

    fn = pl.kernel(body, out_type=out_type, mesh=mesh, scratch_types=scratch,
                   compiler_params=pltpu.CompilerParams(
                       needs_layout_passes=False))
    res = fn(idx, *tables)
    if not isinstance(res, (list, tuple)):
        res = (res,)
    return list(res)


def _readout_body(x_ref, v_ref, mx_ref, sm_ref):
    i = pl.program_id(0)
    xs = x_ref[...] * jnp.tanh(v_ref[...])
    m = jnp.max(xs, axis=0, keepdims=True)
    s = jnp.sum(xs, axis=0, keepdims=True)

    @pl.when(i == 0)
    def _():
        mx_ref[...] = m
        sm_ref[...] = s

    @pl.when(i > 0)
    def _():
        mx_ref[...] = jnp.maximum(mx_ref[...], m)
        sm_ref[...] += s


def _readout(x_gath, vals_col, br):
    n, nhid = x_gath.shape
    return pl.pallas_call(
        _readout_body,
        grid=(n // br,),
        in_specs=[
            pl.BlockSpec((br, nhid), lambda i: (i, 0)),
            pl.BlockSpec((br, 1), lambda i: (i, 0)),
        ],
        out_specs=[
            pl.BlockSpec((1, nhid), lambda i: (0, 0)),
            pl.BlockSpec((1, nhid), lambda i: (0, 0)),
        ],
        out_shape=[
            jax.ShapeDtypeStruct((1, nhid), F32),
            jax.ShapeDtypeStruct((1, nhid), F32),
        ],
    )(x_gath, vals_col)


def kernel(feature, img_info, W_pos, b_pos, g_f, b_f, g_p, b_p,
           W1, b1, W2, b2, W3, b3, Ws1, bs1, Ws2, bs2, Ws3, bs3):
    n = feature.shape[0]
    nf = feature.shape[1]
    nhid = W1.shape[1]
    k1 = math.ceil(0.75 * n)
    k2 = math.ceil(0.75 * k1)
    k3 = math.ceil(0.75 * k2)

    img_p = jnp.pad(img_info, ((0, 0), (0, 128 - img_info.shape[1])))
    wp_p = jnp.pad(W_pos, ((0, 128 - W_pos.shape[0]), (0, 128 - W_pos.shape[1])))
    bp_p = jnp.pad(b_pos, (0, 128 - b_pos.shape[0])).reshape(1, 128)
    feat_p = jnp.pad(feature, ((0, 0), (0, 512 - nf)))
    gf_p = jnp.pad(g_f, (0, 512 - nf)).reshape(1, 512)
    bf_p = jnp.pad(b_f, (0, 512 - nf)).reshape(1, 512)
    gp_p = jnp.pad(g_p, (0, 128 - g_p.shape[0])).reshape(1, 128)
    bpl = jnp.pad(b_p, (0, 128 - b_p.shape[0])).reshape(1, 128)
    w1p = jnp.pad(W1[nf:, :], ((0, 128 - (512 - nf)), (0, 0)))
    ws1_p = jnp.pad(Ws1, ((0, 0), (0, 127)))
    ws2_p = jnp.pad(Ws2, ((0, 0), (0, 127)))
    ws3_p = jnp.pad(Ws3, ((0, 0), (0, 127)))
    bs1_r = jnp.broadcast_to(bs1.reshape(1, 1), (1, 128))
    bs2_r = jnp.broadcast_to(bs2.reshape(1, 1), (1, 128))
    bs3_r = jnp.broadcast_to(bs3.reshape(1, 1), (1, 128))
    b1_r = b1.reshape(1, nhid)
    b2_r = b2.reshape(1, nhid)
    b3_r = b3.reshape(1, nhid)

    lnf, lnp, xw1 = _attn_ln(img_p, wp_p, bp_p, feat_p, gf_p, bf_p, gp_p,
                             bpl, W1, w1p, nhid)
    feat = jnp.concatenate([lnf[:, :nf], lnp[:, :512 - nf]], axis=1)

    d_bf, colmax = _dist_max(feat, 512, 1024)
    t = (0.5 * jnp.max(colmax)).reshape(1, 1)

    mask1, dinv1 = _mask1_deg(d_bf, t, 1024, 1024)
    x1 = _conv(mask1, xw1, dinv1, b1_r, None, None, 1024, 1024)
    s1 = _score(mask1, x1, dinv1, ws1_p, bs1_r, 1024, 1024)
    rank1 = _rank(s1, s1.reshape(1, n))
    perm1, vals1 = _topk_select(rank1.reshape(n), s1.reshape(n), None, k1)
    featsub2, xg1 = _gather_rows(perm1, [feat, x1], k1)

    q2c = perm1.reshape(k1, 1)
    q2r = perm1.reshape(1, k1)
    mask2, dinv2 = _mask23_deg(featsub2, q2c, q2r, t, 1024, 1024)
    x2, mx1, sm1 = _conv(mask2, xg1, dinv2, b2_r, vals1.reshape(k1, 1), W2,
                         1024, 1024)
    read1 = jnp.concatenate([mx1, sm1 / k1], axis=1)
    s2 = _score(mask2, x2, dinv2, ws2_p, bs2_r, 1024, 1024)
    rank2 = _rank(s2, s2.reshape(1, k1))
    perm2, vals2, q3 = _topk_select(rank2.reshape(k1), s2.reshape(k1),
                                    perm1, k2)
    featsub3, xg2 = _gather_rows(perm2, [featsub2, x2], k2)

    q3c = q3.reshape(k2, 1)
    q3r = q3.reshape(1, k2)
    mask3, dinv3 = _mask23_deg(featsub3, q3c, q3r, t, 768, 768)
    x3, mx2, sm2 = _conv(mask3, xg2, dinv3, b3_r, vals2.reshape(k2, 1), W3,
                         768, 768)
    read2 = jnp.concatenate([mx2, sm2 / k2], axis=1)
    s3 = _score(mask3, x3, dinv3, ws3_p, bs3_r, 768, 768)
    rank3 = _rank(s3, s3.reshape(1, k2))
    perm3, vals3 = _topk_select(rank3.reshape(k2), s3.reshape(k2), None, k3)
    (xg3,) = _gather_rows(perm3, [x3], k3)
    mx3, sm3 = _readout(xg3, vals3.reshape(k3, 1), 64)
    read3 = jnp.concatenate([mx3, sm3 / k3], axis=1)

    return read1 + read2 + read3


# speedup vs baseline: 1.4705x; 1.0389x over previous
import functools
import math

import jax
import jax.numpy as jnp
from jax import lax
from jax.experimental import pallas as pl
from jax.experimental.pallas import tpu as pltpu
from jax.experimental.pallas import tpu_sc as plsc

F32 = jnp.float32
BF16 = jnp.bfloat16
I32 = jnp.int32
LN_EPS = 1e-5


def _dot(a, b, ca, cb):
    return lax.dot_general(a, b, ((ca, cb), ((), ())),
                           preferred_element_type=F32)


def _dotb(a, b, ca, cb):
    return lax.dot_general(a.astype(BF16), b.astype(BF16),
                           ((ca, cb), ((), ())), preferred_element_type=F32)


def _attn_ln_body(img_ref, wp_ref, bp_ref, feat_ref, gf_ref, bf_ref, gp_ref,
                  bpl_ref, w1_ref, w1p_ref, lnf_ref, lnp_ref, xw1_ref,
                  pos_scr):
    i = pl.program_id(0)

    @pl.when(i == 0)
    def _():
        pos_scr[...] = _dot(img_ref[...], wp_ref[...], (1,), (0,)) + bp_ref[...]

    br = feat_ref.shape[0]
    pos_all = pos_scr[...]
    pos_blk = pos_scr[pl.ds(i * br, br), :]
    s = _dotb(pos_blk, pos_all, (1,), (1,)) * (1.0 / math.sqrt(12.0))
    m = jnp.max(s, axis=1, keepdims=True)
    p = jnp.exp(s - m)
    den = jnp.sum(p, axis=1, keepdims=True)
    attn = _dotb(p, pos_all, (1,), (0,)) / den
    mu = jnp.sum(attn, axis=1, keepdims=True) / 12.0
    lane = lax.broadcasted_iota(I32, (br, 128), 1)
    xc = jnp.where(lane < 12, attn - mu, 0.0)
    var = jnp.sum(xc * xc, axis=1, keepdims=True) / 12.0
    lnp = xc / jnp.sqrt(var + LN_EPS) * gp_ref[...] + bpl_ref[...]
    f = feat_ref[...]
    muf = jnp.sum(f, axis=1, keepdims=True) / 500.0
    lane2 = lax.broadcasted_iota(I32, (br, 512), 1)
    xcf = jnp.where(lane2 < 500, f - muf, 0.0)
    varf = jnp.sum(xcf * xcf, axis=1, keepdims=True) / 500.0
    lnf = xcf / jnp.sqrt(varf + LN_EPS) * gf_ref[...] + bf_ref[...]
    lnf_ref[...] = lnf
    lnp_ref[...] = lnp
    xw1_ref[...] = (_dot(lnf, w1_ref[...], (1,), (0,))
                    + _dot(lnp, w1p_ref[...], (1,), (0,)))


def _attn_ln(img_p, wp_p, bp_p, feat_p, gf_p, bf_p, gp_p, bpl, w1, w1p, nhid):
    n = img_p.shape[0]
    return pl.pallas_call(
        _attn_ln_body,
        grid=(n // 256,),
        in_specs=[
            pl.BlockSpec((n, 128), lambda i: (0, 0)),
            pl.BlockSpec((128, 128), lambda i: (0, 0)),
            pl.BlockSpec((1, 128), lambda i: (0, 0)),
            pl.BlockSpec((256, 512), lambda i: (i, 0)),
            pl.BlockSpec((1, 512), lambda i: (0, 0)),
            pl.BlockSpec((1, 512), lambda i: (0, 0)),
            pl.BlockSpec((1, 128), lambda i: (0, 0)),
            pl.BlockSpec((1, 128), lambda i: (0, 0)),
            pl.BlockSpec((512, nhid), lambda i: (0, 0)),
            pl.BlockSpec((128, nhid), lambda i: (0, 0)),
        ],
        out_specs=[
            pl.BlockSpec((256, 512), lambda i: (i, 0)),
            pl.BlockSpec((256, 128), lambda i: (i, 0)),
            pl.BlockSpec((256, nhid), lambda i: (i, 0)),
        ],
        out_shape=[
            jax.ShapeDtypeStruct((n, 512), F32),
            jax.ShapeDtypeStruct((n, 128), F32),
            jax.ShapeDtypeStruct((n, nhid), F32),
        ],
        scratch_shapes=[pltpu.VMEM((n, 128), F32)],
    )(img_p, wp_p, bp_p, feat_p, gf_p, bf_p, gp_p, bpl, w1, w1p)


def _dist_body(fi_ref, fj_ref, d_ref, cm_ref):
    i = pl.program_id(1)
    fi = fi_ref[...]
    fj = fj_ref[...]
    sqi = jnp.sum(fi * fi, axis=1, keepdims=True)
    sqj = _dot(jnp.ones((1, 512), F32), fj * fj, (1,), (1,))
    d = sqi + sqj - 2.0 * _dotb(fi, fj, (1,), (1,))
    d_ref[...] = d.astype(BF16)
    cm = jnp.max(d, axis=0, keepdims=True)

    @pl.when(i == 0)
    def _():
        cm_ref[...] = cm

    @pl.when(i > 0)
    def _():
        cm_ref[...] = jnp.maximum(cm_ref[...], cm)


def _dist_max(feat, bi, bj):
    n = feat.shape[0]
    return pl.pallas_call(
        _dist_body,
        grid=(n // bj, n // bi),
        in_specs=[
            pl.BlockSpec((bi, 512), lambda j, i: (i, 0)),
            pl.BlockSpec((bj, 512), lambda j, i: (j, 0)),
        ],
        out_specs=[
            pl.BlockSpec((bi, bj), lambda j, i: (i, j)),
            pl.BlockSpec((1, bj), lambda j, i: (0, j)),
        ],
        out_shape=[
            jax.ShapeDtypeStruct((n, n), BF16),
            jax.ShapeDtypeStruct((1, n), F32),
        ],
    )(feat, feat)


def _mask1_body(d_ref, t_ref, mask_ref, dinv_ref):
    j = pl.program_id(1)
    i = pl.program_id(0)
    nj = pl.num_programs(1)
    bi, bj = d_ref.shape
    t = t_ref[0, 0]
    gi = lax.broadcasted_iota(I32, (bi, bj), 0) + i * bi
    gj = lax.broadcasted_iota(I32, (bi, bj), 1) + j * bj
    cond = (d_ref[...].astype(F32) < t) & (gj < gi)
    m = jnp.where(cond, 1.0, 0.0).astype(BF16)
    mask_ref[...] = m
    degp = _dot(m, jnp.ones((bj, 128), BF16), (1,), (0,))[:, :1]

    @pl.when(j == 0)
    def _():
        dinv_ref[...] = 1.0 + degp

    @pl.when(j > 0)
    def _():
        dinv_ref[...] += degp

    @pl.when(j == nj - 1)
    def _():
        dinv_ref[...] = 1.0 / jnp.sqrt(dinv_ref[...])


def _mask1_deg(d_bf, t, bi, bj):
    n = d_bf.shape[0]
    return pl.pallas_call(
        _mask1_body,
        grid=(n // bi, n // bj),
        in_specs=[
            pl.BlockSpec((bi, bj), lambda i, j: (i, j)),
            pl.BlockSpec(memory_space=pltpu.SMEM),
        ],
        out_specs=[
            pl.BlockSpec((bi, bj), lambda i, j: (i, j)),
            pl.BlockSpec((bi, 1), lambda i, j: (i, 0)),
        ],
        out_shape=[
            jax.ShapeDtypeStruct((n, n), BF16),
            jax.ShapeDtypeStruct((n, 1), F32),
        ],
    )(d_bf, t)


def _mask23_body(fi_ref, fj_ref, qc_ref, qr_ref, t_ref, mask_ref, dinv_ref):
    j = pl.program_id(1)
    nj = pl.num_programs(1)
    bi = fi_ref.shape[0]
    bj = fj_ref.shape[0]
    fi = fi_ref[...]
    fj = fj_ref[...]
    sqi = jnp.sum(fi * fi, axis=1, keepdims=True)
    sqj = _dot(jnp.ones((1, 512), F32), fj * fj, (1,), (1,))
    d = sqi + sqj - 2.0 * _dotb(fi, fj, (1,), (1,))
    t = t_ref[0, 0]
    cond = (d < t) & (qr_ref[...] < qc_ref[...])
    m = jnp.where(cond, 1.0, 0.0).astype(BF16)
    mask_ref[...] = m
    degp = _dot(m, jnp.ones((bj, 128), BF16), (1,), (0,))[:, :1]

    @pl.when(j == 0)
    def _():
        dinv_ref[...] = 1.0 + degp

    @pl.when(j > 0)
    def _():
        dinv_ref[...] += degp

    @pl.when(j == nj - 1)
    def _():
        dinv_ref[...] = 1.0 / jnp.sqrt(dinv_ref[...])


def _mask23_deg(featsub, qc, qr, t, bi, bj):
    n = featsub.shape[0]
    return pl.pallas_call(
        _mask23_body,
        grid=(n // bi, n // bj),
        in_specs=[
            pl.BlockSpec((bi, 512), lambda i, j: (i, 0)),
            pl.BlockSpec((bj, 512), lambda i, j: (j, 0)),
            pl.BlockSpec((bi, 1), lambda i, j: (i, 0)),
            pl.BlockSpec((1, bj), lambda i, j: (0, j)),
            pl.BlockSpec(memory_space=pltpu.SMEM),
        ],
        out_specs=[
            pl.BlockSpec((bi, bj), lambda i, j: (i, j)),
            pl.BlockSpec((bi, 1), lambda i, j: (i, 0)),
        ],
        out_shape=[
            jax.ShapeDtypeStruct((n, n), BF16),
            jax.ShapeDtypeStruct((n, 1), F32),
        ],
    )(featsub, featsub, qc, qr, t)


def _conv_body(fused, mask_ref, x_ref, dinvj_ref, dinvi_ref, b_ref, *rest):
    if fused:
        v_ref, w_ref, out_ref, mx_ref, sm_ref = rest
    else:
        out_ref, = rest
    i = pl.program_id(0)
    j = pl.program_id(1)
    nj = pl.num_programs(1)
    bi = mask_ref.shape[0]
    bj = mask_ref.shape[1]
    if fused:
        xs = x_ref[...] * jnp.tanh(v_ref[...])
        xw = _dot(xs, w_ref[...], (1,), (0,))

        @pl.when(i == 0)
        def _():
            mro = jnp.max(xs, axis=0, keepdims=True)
            sro = jnp.sum(xs, axis=0, keepdims=True)

            @pl.when(j == 0)
            def _():
                mx_ref[...] = mro
                sm_ref[...] = sro

            @pl.when(j > 0)
            def _():
                mx_ref[...] = jnp.maximum(mx_ref[...], mro)
                sm_ref[...] += sro
    else:
        xw = x_ref[...]
    z = xw * dinvj_ref[...]
    contrib = _dotb(mask_ref[...], z, (1,), (0,))

    @pl.when(j == 0)
    def _():
        out_ref[...] = contrib

    @pl.when(j > 0)
    def _():
        out_ref[...] += contrib

    off = j * bj - i * bi

    @pl.when((off >= 0) & (off < bi))
    def _():
        out_ref[pl.ds(pl.multiple_of(off, bj), bj), :] += z

    @pl.when(j == nj - 1)
    def _():
        out_ref[...] = jnp.maximum(out_ref[...] * dinvi_ref[...] + b_ref[...],
                                   0.0)


def _conv(maskT, xin, dinv, b_row, vals_col, w_next, bi, bj):
    n = maskT.shape[0]
    nhid = xin.shape[1]
    fused = vals_col is not None
    in_specs = [
        pl.BlockSpec((bi, bj), lambda i, j: (i, j)),
        pl.BlockSpec((bj, nhid), lambda i, j: (j, 0)),
        pl.BlockSpec((bj, 1), lambda i, j: (j, 0)),
        pl.BlockSpec((bi, 1), lambda i, j: (i, 0)),
        pl.BlockSpec((1, nhid), lambda i, j: (0, 0)),
    ]
    args = [maskT, xin, dinv, dinv, b_row]
    out_specs = [pl.BlockSpec((bi, nhid), lambda i, j: (i, 0))]
    out_shape = [jax.ShapeDtypeStruct((n, nhid), F32)]
    if fused:
        in_specs += [
            pl.BlockSpec((bj, 1), lambda i, j: (j, 0)),
            pl.BlockSpec((nhid, nhid), lambda i, j: (0, 0)),
        ]
        args += [vals_col, w_next]
        out_specs += [
            pl.BlockSpec((1, nhid), lambda i, j: (0, 0)),
            pl.BlockSpec((1, nhid), lambda i, j: (0, 0)),
        ]
        out_shape += [
            jax.ShapeDtypeStruct((1, nhid), F32),
            jax.ShapeDtypeStruct((1, nhid), F32),
        ]
    res = pl.pallas_call(
        functools.partial(_conv_body, fused),
        grid=(n // bi, n // bj),
        in_specs=in_specs,
        out_specs=out_specs,
        out_shape=out_shape,
    )(*args)
    return res if fused else res[0]


def _score_body(mask_ref, x_ref, dinvj_ref, dinvi_ref, ws_ref, bs_ref,
                out_ref):
    i = pl.program_id(0)
    j = pl.program_id(1)
    nj = pl.num_programs(1)
    bi = mask_ref.shape[0]
    bj = mask_ref.shape[1]
    u = _dot(x_ref[...], ws_ref[...], (1,), (0,)) * dinvj_ref[...]
    contrib = _dotb(mask_ref[...], u, (1,), (0,))

    @pl.when(j == 0)
    def _():
        out_ref[...] = contrib

    @pl.when(j > 0)
    def _():
        out_ref[...] += contrib

    off = j * bj - i * bi

    @pl.when((off >= 0) & (off < bi))
    def _():
        out_ref[pl.ds(pl.multiple_of(off, bj), bj), :] += u

    @pl.when(j == nj - 1)
    def _():
        out_ref[...] = out_ref[...] * dinvi_ref[...] + bs_ref[...]


def _score(maskT, x, dinv, ws_p, bs_row, bi, bj):
    n = maskT.shape[0]
    nhid = x.shape[1]
    out = pl.pallas_call(
        _score_body,
        grid=(n // bi, n // bj),
        in_specs=[
            pl.BlockSpec((bi, bj), lambda i, j: (i, j)),
            pl.BlockSpec((bj, nhid), lambda i, j: (j, 0)),
            pl.BlockSpec((bj, 1), lambda i, j: (j, 0)),
            pl.BlockSpec((bi, 1), lambda i, j: (i, 0)),
            pl.BlockSpec((nhid, 128), lambda i, j: (0, 0)),
            pl.BlockSpec((1, 128), lambda i, j: (0, 0)),
        ],
        out_specs=pl.BlockSpec((bi, 128), lambda i, j: (i, 0)),
        out_shape=jax.ShapeDtypeStruct((n, 128), F32),
    )(maskT, x, dinv, dinv, ws_p, bs_row)
    return out[:, :1]


def _rank_body(sc_ref, sr_ref, rank_ref):
    i = pl.program_id(0)
    bi = sc_ref.shape[0]
    n = sr_ref.shape[1]
    si = sc_ref[...]
    sj = sr_ref[...]
    gi = lax.broadcasted_iota(I32, (bi, n), 0) + i * bi
    gj = lax.broadcasted_iota(I32, (bi, n), 1)
    before = (sj > si) | ((sj == si) & (gj < gi))
    rank_ref[...] = jnp.sum(before.astype(I32), axis=1, keepdims=True)


def _rank(sc, sr):
    n = sc.shape[0]
    bi = 256
    return pl.pallas_call(
        _rank_body,
        grid=(n // bi,),
        in_specs=[
            pl.BlockSpec((bi, 1), lambda i: (i, 0)),
            pl.BlockSpec((1, n), lambda i: (0, 0)),
        ],
        out_specs=pl.BlockSpec((bi, 1), lambda i: (i, 0)),
        out_shape=jax.ShapeDtypeStruct((n, 1), I32),
    )(sc, sr)


def _select_gather(rank_flat, s_flat, q_flat, k_out, tables):
    k_in = rank_flat.shape[0]
    with_q = q_flat is not None
    nt = len(tables)
    n_workers = 32
    c = k_out // n_workers
    if c % 8 != 0:
        c = 64
        n_workers = k_out // c
    assert c * n_workers == k_out and c % 8 == 0 and c <= 128
    mesh = plsc.VectorSubcoreMesh(core_axis_name="c", subcore_axis_name="s")
    out_type = [jax.ShapeDtypeStruct((k_out,), I32),
                jax.ShapeDtypeStruct((k_out,), F32)]
    if with_q:
        out_type.append(jax.ShapeDtypeStruct((k_out,), I32))
    out_type += [jax.ShapeDtypeStruct((k_out, tb.shape[1]), F32)
                 for tb in tables]
    scratch = [pltpu.VMEM((k_in,), I32), pltpu.VMEM((k_in,), F32),
               pltpu.VMEM((k_out,), I32), pltpu.VMEM((k_out,), F32)]
    if with_q:
        scratch += [pltpu.VMEM((k_in,), I32), pltpu.VMEM((k_out,), I32)]
    scratch += [pltpu.VMEM_SHARED((k_out,), I32), pltpu.VMEM((c,), I32)]
    scratch += [pltpu.VMEM((c, tb.shape[1]), F32) for tb in tables]
    scratch += [pltpu.SemaphoreType.DMA]

    def body(*refs):
        pos = 0
        rank_hbm, s_hbm = refs[0], refs[1]
        pos = 2
        if with_q:
            q_hbm = refs[pos]
            pos += 1
        tabs = refs[pos:pos + nt]
        pos += nt
        perm_out, vals_out = refs[pos], refs[pos + 1]
        pos += 2
        if with_q:
            q_out = refs[pos]
            pos += 1
        outs = refs[pos:pos + nt]
        pos += nt
        rank_v, s_v, perm_v, vals_v = refs[pos:pos + 4]
        pos += 4
        if with_q:
            q_v, qn_v = refs[pos:pos + 2]
            pos += 2
        perm_sh, idx_v = refs[pos], refs[pos + 1]
        pos += 2
        bufs = refs[pos:pos + nt]
        sem = refs[pos + nt]

        cid = lax.axis_index("c")
        sid = lax.axis_index("s")

        @pl.when(sid == 0)
        def _():
            pltpu.sync_copy(rank_hbm, rank_v)
            pltpu.sync_copy(s_hbm, s_v)
            if with_q:
                pltpu.sync_copy(q_hbm, q_v)

            def step(cc, carry):
                base = cc * 16
                idx = rank_v[pl.ds(base, 16)]
                msk = idx < k_out
                ids = lax.iota(I32, 16) + base
                plsc.store_scatter(perm_v, [idx], ids, mask=msk)
                plsc.store_scatter(vals_v, [idx], s_v[pl.ds(base, 16)],
                                   mask=msk)
                if with_q:
                    plsc.store_scatter(qn_v, [idx], q_v[pl.ds(base, 16)],
                                       mask=msk)
                return carry

            lax.fori_loop(0, k_in // 16, step, 0)
            pltpu.sync_copy(perm_v, perm_sh)

            @pl.when(cid == 0)
            def _():
                pltpu.sync_copy(perm_v, perm_out)
                pltpu.sync_copy(vals_v, vals_out)
                if with_q:
                    pltpu.sync_copy(qn_v, q_out)

        plsc.subcore_barrier()
        wid = sid * 2 + cid

        @pl.when(wid < n_workers)
        def _():
            base = wid * c
            pltpu.sync_copy(perm_sh.at[pl.ds(base, c)], idx_v)
            for tb, buf, out in zip(tabs, bufs, outs):
                pltpu.async_copy(tb.at[idx_v], buf, sem).wait()
                pltpu.sync_copy(buf, out.at[pl.ds(base, c)])

    fn = pl.kernel(body, out_type=tuple(out_type), mesh=mesh,
                   scratch_types=tuple(scratch),
                   compiler_params=pltpu.CompilerParams(
                       needs_layout_passes=False))
    if with_q:
        return fn(rank_flat, s_flat, q_flat, *tables)
    return fn(rank_flat, s_flat, *tables)


def _readout_body(x_ref, v_ref, mx_ref, sm_ref):
    i = pl.program_id(0)
    xs = x_ref[...] * jnp.tanh(v_ref[...])
    m = jnp.max(xs, axis=0, keepdims=True)
    s = jnp.sum(xs, axis=0, keepdims=True)

    @pl.when(i == 0)
    def _():
        mx_ref[...] = m
        sm_ref[...] = s

    @pl.when(i > 0)
    def _():
        mx_ref[...] = jnp.maximum(mx_ref[...], m)
        sm_ref[...] += s


def _readout(x_gath, vals_col, br):
    n, nhid = x_gath.shape
    return pl.pallas_call(
        _readout_body,
        grid=(n // br,),
        in_specs=[
            pl.BlockSpec((br, nhid), lambda i: (i, 0)),
            pl.BlockSpec((br, 1), lambda i: (i, 0)),
        ],
        out_specs=[
            pl.BlockSpec((1, nhid), lambda i: (0, 0)),
            pl.BlockSpec((1, nhid), lambda i: (0, 0)),
        ],
        out_shape=[
            jax.ShapeDtypeStruct((1, nhid), F32),
            jax.ShapeDtypeStruct((1, nhid), F32),
        ],
    )(x_gath, vals_col)


def kernel(feature, img_info, W_pos, b_pos, g_f, b_f, g_p, b_p,
           W1, b1, W2, b2, W3, b3, Ws1, bs1, Ws2, bs2, Ws3, bs3):
    n = feature.shape[0]
    nf = feature.shape[1]
    nhid = W1.shape[1]
    k1 = math.ceil(0.75 * n)
    k2 = math.ceil(0.75 * k1)
    k3 = math.ceil(0.75 * k2)

    img_p = jnp.pad(img_info, ((0, 0), (0, 128 - img_info.shape[1])))
    wp_p = jnp.pad(W_pos, ((0, 128 - W_pos.shape[0]), (0, 128 - W_pos.shape[1])))
    bp_p = jnp.pad(b_pos, (0, 128 - b_pos.shape[0])).reshape(1, 128)
    feat_p = jnp.pad(feature, ((0, 0), (0, 512 - nf)))
    gf_p = jnp.pad(g_f, (0, 512 - nf)).reshape(1, 512)
    bf_p = jnp.pad(b_f, (0, 512 - nf)).reshape(1, 512)
    gp_p = jnp.pad(g_p, (0, 128 - g_p.shape[0])).reshape(1, 128)
    bpl = jnp.pad(b_p, (0, 128 - b_p.shape[0])).reshape(1, 128)
    w1p = jnp.pad(W1[nf:, :], ((0, 128 - (512 - nf)), (0, 0)))
    ws1_p = jnp.pad(Ws1, ((0, 0), (0, 127)))
    ws2_p = jnp.pad(Ws2, ((0, 0), (0, 127)))
    ws3_p = jnp.pad(Ws3, ((0, 0), (0, 127)))
    bs1_r = jnp.broadcast_to(bs1.reshape(1, 1), (1, 128))
    bs2_r = jnp.broadcast_to(bs2.reshape(1, 1), (1, 128))
    bs3_r = jnp.broadcast_to(bs3.reshape(1, 1), (1, 128))
    b1_r = b1.reshape(1, nhid)
    b2_r = b2.reshape(1, nhid)
    b3_r = b3.reshape(1, nhid)

    lnf, lnp, xw1 = _attn_ln(img_p, wp_p, bp_p, feat_p, gf_p, bf_p, gp_p,
                             bpl, W1, w1p, nhid)
    feat = jnp.concatenate([lnf[:, :nf], lnp[:, :512 - nf]], axis=1)

    d_bf, colmax = _dist_max(feat, 512, 1024)
    t = (0.5 * jnp.max(colmax)).reshape(1, 1)

    mask1, dinv1 = _mask1_deg(d_bf, t, 1024, 1024)
    x1 = _conv(mask1, xw1, dinv1, b1_r, None, None, 1024, 1024)
    s1 = _score(mask1, x1, dinv1, ws1_p, bs1_r, 1024, 1024)
    rank1 = _rank(s1, s1.reshape(1, n))
    perm1, vals1, featsub2, xg1 = _select_gather(
        rank1.reshape(n), s1.reshape(n), None, k1, [feat, x1])

    q2c = perm1.reshape(k1, 1)
    q2r = perm1.reshape(1, k1)
    mask2, dinv2 = _mask23_deg(featsub2, q2c, q2r, t, 1024, 1024)
    x2, mx1, sm1 = _conv(mask2, xg1, dinv2, b2_r, vals1.reshape(k1, 1), W2,
                         1024, 1024)
    read1 = jnp.concatenate([mx1, sm1 / k1], axis=1)
    s2 = _score(mask2, x2, dinv2, ws2_p, bs2_r, 1024, 1024)
    rank2 = _rank(s2, s2.reshape(1, k1))
    perm2, vals2, q3, featsub3, xg2 = _select_gather(
        rank2.reshape(k1), s2.reshape(k1), perm1, k2, [featsub2, x2])

    q3c = q3.reshape(k2, 1)
    q3r = q3.reshape(1, k2)
    mask3, dinv3 = _mask23_deg(featsub3, q3c, q3r, t, 768, 768)
    x3, mx2, sm2 = _conv(mask3, xg2, dinv3, b3_r, vals2.reshape(k2, 1), W3,
                         768, 768)
    read2 = jnp.concatenate([mx2, sm2 / k2], axis=1)
    s3 = _score(mask3, x3, dinv3, ws3_p, bs3_r, 768, 768)
    rank3 = _rank(s3, s3.reshape(1, k2))
    perm3, vals3, xg3 = _select_gather(
        rank3.reshape(k2), s3.reshape(k2), None, k3, [x3])
    mx3, sm3 = _readout(xg3, vals3.reshape(k3, 1), 192)
    read3 = jnp.concatenate([mx3, sm3 / k3], axis=1)

    return read1 + read2 + read3


# speedup vs baseline: 1.4717x; 1.0008x over previous
import functools
import math

import jax
import jax.numpy as jnp
from jax import lax
from jax.experimental import pallas as pl
from jax.experimental.pallas import tpu as pltpu
from jax.experimental.pallas import tpu_sc as plsc

F32 = jnp.float32
BF16 = jnp.bfloat16
I32 = jnp.int32
LN_EPS = 1e-5


def _dot(a, b, ca, cb):
    return lax.dot_general(a, b, ((ca, cb), ((), ())),
                           preferred_element_type=F32)


def _dotb(a, b, ca, cb):
    return lax.dot_general(a.astype(BF16), b.astype(BF16),
                           ((ca, cb), ((), ())), preferred_element_type=F32)


def _attn_ln_body(img_ref, wp_ref, bp_ref, feat_ref, gf_ref, bf_ref, gp_ref,
                  bpl_ref, w1_ref, w1p_ref, lnf_ref, lnp_ref, xw1_ref,
                  pos_scr):
    i = pl.program_id(0)

    @pl.when(i == 0)
    def _():
        pos_scr[...] = _dot(img_ref[...], wp_ref[...], (1,), (0,)) + bp_ref[...]

    br = feat_ref.shape[0]
    pos_all = pos_scr[...]
    pos_blk = pos_scr[pl.ds(i * br, br), :]
    s = _dotb(pos_blk, pos_all, (1,), (1,)) * (1.0 / math.sqrt(12.0))
    m = jnp.max(s, axis=1, keepdims=True)
    p = jnp.exp(s - m)
    den = jnp.sum(p, axis=1, keepdims=True)
    attn = _dotb(p, pos_all, (1,), (0,)) / den
    mu = jnp.sum(attn, axis=1, keepdims=True) / 12.0
    lane = lax.broadcasted_iota(I32, (br, 128), 1)
    xc = jnp.where(lane < 12, attn - mu, 0.0)
    var = jnp.sum(xc * xc, axis=1, keepdims=True) / 12.0
    lnp = xc / jnp.sqrt(var + LN_EPS) * gp_ref[...] + bpl_ref[...]
    f = feat_ref[...]
    muf = jnp.sum(f, axis=1, keepdims=True) / 500.0
    lane2 = lax.broadcasted_iota(I32, (br, 512), 1)
    xcf = jnp.where(lane2 < 500, f - muf, 0.0)
    varf = jnp.sum(xcf * xcf, axis=1, keepdims=True) / 500.0
    lnf = xcf / jnp.sqrt(varf + LN_EPS) * gf_ref[...] + bf_ref[...]
    lnf_ref[...] = lnf
    lnp_ref[...] = lnp
    xw1_ref[...] = (_dot(lnf, w1_ref[...], (1,), (0,))
                    + _dot(lnp, w1p_ref[...], (1,), (0,)))


def _attn_ln(img_p, wp_p, bp_p, feat_p, gf_p, bf_p, gp_p, bpl, w1, w1p, nhid):
    n = img_p.shape[0]
    return pl.pallas_call(
        _attn_ln_body,
        grid=(n // 256,),
        in_specs=[
            pl.BlockSpec((n, 128), lambda i: (0, 0)),
            pl.BlockSpec((128, 128), lambda i: (0, 0)),
            pl.BlockSpec((1, 128), lambda i: (0, 0)),
            pl.BlockSpec((256, 512), lambda i: (i, 0)),
            pl.BlockSpec((1, 512), lambda i: (0, 0)),
            pl.BlockSpec((1, 512), lambda i: (0, 0)),
            pl.BlockSpec((1, 128), lambda i: (0, 0)),
            pl.BlockSpec((1, 128), lambda i: (0, 0)),
            pl.BlockSpec((512, nhid), lambda i: (0, 0)),
            pl.BlockSpec((128, nhid), lambda i: (0, 0)),
        ],
        out_specs=[
            pl.BlockSpec((256, 512), lambda i: (i, 0)),
            pl.BlockSpec((256, 128), lambda i: (i, 0)),
            pl.BlockSpec((256, nhid), lambda i: (i, 0)),
        ],
        out_shape=[
            jax.ShapeDtypeStruct((n, 512), F32),
            jax.ShapeDtypeStruct((n, 128), F32),
            jax.ShapeDtypeStruct((n, nhid), F32),
        ],
        scratch_shapes=[pltpu.VMEM((n, 128), F32)],
    )(img_p, wp_p, bp_p, feat_p, gf_p, bf_p, gp_p, bpl, w1, w1p)


def _dist_body(fi_ref, fj_ref, d_ref, cm_ref):
    i = pl.program_id(1)
    fi = fi_ref[...]
    fj = fj_ref[...]
    sqi = jnp.sum(fi * fi, axis=1, keepdims=True)
    sqj = _dot(jnp.ones((1, 512), F32), fj * fj, (1,), (1,))
    d = sqi + sqj - 2.0 * _dotb(fi, fj, (1,), (1,))
    d_ref[...] = d.astype(BF16)
    cm = jnp.max(d, axis=0, keepdims=True)

    @pl.when(i == 0)
    def _():
        cm_ref[...] = cm

    @pl.when(i > 0)
    def _():
        cm_ref[...] = jnp.maximum(cm_ref[...], cm)


def _dist_max(feat, bi, bj):
    n = feat.shape[0]
    return pl.pallas_call(
        _dist_body,
        grid=(n // bj, n // bi),
        in_specs=[
            pl.BlockSpec((bi, 512), lambda j, i: (i, 0)),
            pl.BlockSpec((bj, 512), lambda j, i: (j, 0)),
        ],
        out_specs=[
            pl.BlockSpec((bi, bj), lambda j, i: (i, j)),
            pl.BlockSpec((1, bj), lambda j, i: (0, j)),
        ],
        out_shape=[
            jax.ShapeDtypeStruct((n, n), BF16),
            jax.ShapeDtypeStruct((1, n), F32),
        ],
    )(feat, feat)


def _mask1_body(d_ref, t_ref, mask_ref, dinv_ref):
    j = pl.program_id(1)
    i = pl.program_id(0)
    nj = pl.num_programs(1)
    bi, bj = d_ref.shape
    t = 0.5 * jnp.max(t_ref[...])
    gi = lax.broadcasted_iota(I32, (bi, bj), 0) + i * bi
    gj = lax.broadcasted_iota(I32, (bi, bj), 1) + j * bj
    cond = (d_ref[...].astype(F32) < t) & (gj < gi)
    m = jnp.where(cond, 1.0, 0.0).astype(BF16)
    mask_ref[...] = m
    degp = _dot(m, jnp.ones((bj, 128), BF16), (1,), (0,))[:, :1]

    @pl.when(j == 0)
    def _():
        dinv_ref[...] = 1.0 + degp

    @pl.when(j > 0)
    def _():
        dinv_ref[...] += degp

    @pl.when(j == nj - 1)
    def _():
        dinv_ref[...] = 1.0 / jnp.sqrt(dinv_ref[...])


def _mask1_deg(d_bf, t, bi, bj):
    n = d_bf.shape[0]
    return pl.pallas_call(
        _mask1_body,
        grid=(n // bi, n // bj),
        in_specs=[
            pl.BlockSpec((bi, bj), lambda i, j: (i, j)),
            pl.BlockSpec((1, t.shape[1]), lambda i, j: (0, 0)),
        ],
        out_specs=[
            pl.BlockSpec((bi, bj), lambda i, j: (i, j)),
            pl.BlockSpec((bi, 1), lambda i, j: (i, 0)),
        ],
        out_shape=[
            jax.ShapeDtypeStruct((n, n), BF16),
            jax.ShapeDtypeStruct((n, 1), F32),
        ],
    )(d_bf, t)


def _mask23_body(fi_ref, fj_ref, qc_ref, qr_ref, t_ref, mask_ref, dinv_ref):
    j = pl.program_id(1)
    nj = pl.num_programs(1)
    bi = fi_ref.shape[0]
    bj = fj_ref.shape[0]
    fi = fi_ref[...]
    fj = fj_ref[...]
    sqi = jnp.sum(fi * fi, axis=1, keepdims=True)
    sqj = _dot(jnp.ones((1, 512), F32), fj * fj, (1,), (1,))
    d = sqi + sqj - 2.0 * _dotb(fi, fj, (1,), (1,))
    t = 0.5 * jnp.max(t_ref[...])
    cond = (d < t) & (qr_ref[...] < qc_ref[...])
    m = jnp.where(cond, 1.0, 0.0).astype(BF16)
    mask_ref[...] = m
    degp = _dot(m, jnp.ones((bj, 128), BF16), (1,), (0,))[:, :1]

    @pl.when(j == 0)
    def _():
        dinv_ref[...] = 1.0 + degp

    @pl.when(j > 0)
    def _():
        dinv_ref[...] += degp

    @pl.when(j == nj - 1)
    def _():
        dinv_ref[...] = 1.0 / jnp.sqrt(dinv_ref[...])


def _mask23_deg(featsub, qc, qr, t, bi, bj):
    n = featsub.shape[0]
    return pl.pallas_call(
        _mask23_body,
        grid=(n // bi, n // bj),
        in_specs=[
            pl.BlockSpec((bi, 512), lambda i, j: (i, 0)),
            pl.BlockSpec((bj, 512), lambda i, j: (j, 0)),
            pl.BlockSpec((bi, 1), lambda i, j: (i, 0)),
            pl.BlockSpec((1, bj), lambda i, j: (0, j)),
            pl.BlockSpec((1, t.shape[1]), lambda i, j: (0, 0)),
        ],
        out_specs=[
            pl.BlockSpec((bi, bj), lambda i, j: (i, j)),
            pl.BlockSpec((bi, 1), lambda i, j: (i, 0)),
        ],
        out_shape=[
            jax.ShapeDtypeStruct((n, n), BF16),
            jax.ShapeDtypeStruct((n, 1), F32),
        ],
    )(featsub, featsub, qc, qr, t)


def _conv_body(fused, mask_ref, x_ref, dinvj_ref, dinvi_ref, b_ref, *rest):
    if fused:
        v_ref, w_ref, out_ref, mx_ref, sm_ref = rest
    else:
        out_ref, = rest
    i = pl.program_id(0)
    j = pl.program_id(1)
    nj = pl.num_programs(1)
    bi = mask_ref.shape[0]
    bj = mask_ref.shape[1]
    if fused:
        xs = x_ref[...] * jnp.tanh(v_ref[...])
        xw = _dot(xs, w_ref[...], (1,), (0,))

        @pl.when(i == 0)
        def _():
            mro = jnp.max(xs, axis=0, keepdims=True)
            sro = jnp.sum(xs, axis=0, keepdims=True)

            @pl.when(j == 0)
            def _():
                mx_ref[...] = mro
                sm_ref[...] = sro

            @pl.when(j > 0)
            def _():
                mx_ref[...] = jnp.maximum(mx_ref[...], mro)
                sm_ref[...] += sro
    else:
        xw = x_ref[...]
    z = xw * dinvj_ref[...]
    contrib = _dotb(mask_ref[...], z, (1,), (0,))

    @pl.when(j == 0)
    def _():
        out_ref[...] = contrib

    @pl.when(j > 0)
    def _():
        out_ref[...] += contrib

    off = j * bj - i * bi

    @pl.when((off >= 0) & (off < bi))
    def _():
        out_ref[pl.ds(pl.multiple_of(off, bj), bj), :] += z

    @pl.when(j == nj - 1)
    def _():
        out_ref[...] = jnp.maximum(out_ref[...] * dinvi_ref[...] + b_ref[...],
                                   0.0)


def _conv(maskT, xin, dinv, b_row, vals_col, w_next, bi, bj):
    n = maskT.shape[0]
    nhid = xin.shape[1]
    fused = vals_col is not None
    in_specs = [
        pl.BlockSpec((bi, bj), lambda i, j: (i, j)),
        pl.BlockSpec((bj, nhid), lambda i, j: (j, 0)),
        pl.BlockSpec((bj, 1), lambda i, j: (j, 0)),
        pl.BlockSpec((bi, 1), lambda i, j: (i, 0)),
        pl.BlockSpec((1, nhid), lambda i, j: (0, 0)),
    ]
    args = [maskT, xin, dinv, dinv, b_row]
    out_specs = [pl.BlockSpec((bi, nhid), lambda i, j: (i, 0))]
    out_shape = [jax.ShapeDtypeStruct((n, nhid), F32)]
    if fused:
        in_specs += [
            pl.BlockSpec((bj, 1), lambda i, j: (j, 0)),
            pl.BlockSpec((nhid, nhid), lambda i, j: (0, 0)),
        ]
        args += [vals_col, w_next]
        out_specs += [
            pl.BlockSpec((1, nhid), lambda i, j: (0, 0)),
            pl.BlockSpec((1, nhid), lambda i, j: (0, 0)),
        ]
        out_shape += [
            jax.ShapeDtypeStruct((1, nhid), F32),
            jax.ShapeDtypeStruct((1, nhid), F32),
        ]
    res = pl.pallas_call(
        functools.partial(_conv_body, fused),
        grid=(n // bi, n // bj),
        in_specs=in_specs,
        out_specs=out_specs,
        out_shape=out_shape,
    )(*args)
    return res if fused else res[0]


def _score_body(mask_ref, x_ref, dinvj_ref, dinvi_ref, ws_ref, bs_ref,
                out_ref):
    i = pl.program_id(0)
    j = pl.program_id(1)
    nj = pl.num_programs(1)
    bi = mask_ref.shape[0]
    bj = mask_ref.shape[1]
    u = _dot(x_ref[...], ws_ref[...], (1,), (0,)) * dinvj_ref[...]
    contrib = _dotb(mask_ref[...], u, (1,), (0,))

    @pl.when(j == 0)
    def _():
        out_ref[...] = contrib

    @pl.when(j > 0)
    def _():
        out_ref[...] += contrib

    off = j * bj - i * bi

    @pl.when((off >= 0) & (off < bi))
    def _():
        out_ref[pl.ds(pl.multiple_of(off, bj), bj), :] += u

    @pl.when(j == nj - 1)
    def _():
        out_ref[...] = out_ref[...] * dinvi_ref[...] + bs_ref[...]


def _score(maskT, x, dinv, ws_p, bs_row, bi, bj):
    n = maskT.shape[0]
    nhid = x.shape[1]
    out = pl.pallas_call(
        _score_body,
        grid=(n // bi, n // bj),
        in_specs=[
            pl.BlockSpec((bi, bj), lambda i, j: (i, j)),
            pl.BlockSpec((bj, nhid), lambda i, j: (j, 0)),
            pl.BlockSpec((bj, 1), lambda i, j: (j, 0)),
            pl.BlockSpec((bi, 1), lambda i, j: (i, 0)),
            pl.BlockSpec((nhid, 128), lambda i, j: (0, 0)),
            pl.BlockSpec((1, 128), lambda i, j: (0, 0)),
        ],
        out_specs=pl.BlockSpec((bi, 128), lambda i, j: (i, 0)),
        out_shape=jax.ShapeDtypeStruct((n, 128), F32),
    )(maskT, x, dinv, dinv, ws_p, bs_row)
    return out[:, :1]


def _rank_body(sc_ref, sr_ref, rank_ref):
    i = pl.program_id(0)
    bi = sc_ref.shape[0]
    n = sr_ref.shape[1]
    si = sc_ref[...]
    sj = sr_ref[...]
    gi = lax.broadcasted_iota(I32, (bi, n), 0) + i * bi
    gj = lax.broadcasted_iota(I32, (bi, n), 1)
    before = (sj > si) | ((sj == si) & (gj < gi))
    rank_ref[...] = jnp.sum(before.astype(I32), axis=1, keepdims=True)


def _rank(sc, sr):
    n = sc.shape[0]
    bi = 256
    return pl.pallas_call(
        _rank_body,
        grid=(n // bi,),
        in_specs=[
            pl.BlockSpec((bi, 1), lambda i: (i, 0)),
            pl.BlockSpec((1, n), lambda i: (0, 0)),
        ],
        out_specs=pl.BlockSpec((bi, 1), lambda i: (i, 0)),
        out_shape=jax.ShapeDtypeStruct((n, 1), I32),
    )(sc, sr)


def _select_gather(rank_flat, s_flat, q_flat, k_out, tables):
    k_in = rank_flat.shape[0]
    with_q = q_flat is not None
    nt = len(tables)
    n_workers = 32
    c = k_out // n_workers
    if c % 8 != 0:
        c = 64
        n_workers = k_out // c
    assert c * n_workers == k_out and c % 8 == 0 and c <= 128
    mesh = plsc.VectorSubcoreMesh(core_axis_name="c", subcore_axis_name="s")
    out_type = [jax.ShapeDtypeStruct((k_out,), I32),
                jax.ShapeDtypeStruct((k_out,), F32)]
    if with_q:
        out_type.append(jax.ShapeDtypeStruct((k_out,), I32))
    out_type += [jax.ShapeDtypeStruct((k_out, tb.shape[1]), F32)
                 for tb in tables]
    scratch = [pltpu.VMEM((k_in,), I32), pltpu.VMEM((k_in,), F32),
               pltpu.VMEM((k_out,), I32), pltpu.VMEM((k_out,), F32)]
    if with_q:
        scratch += [pltpu.VMEM((k_in,), I32), pltpu.VMEM((k_out,), I32)]
    scratch += [pltpu.VMEM_SHARED((k_out,), I32), pltpu.VMEM((c,), I32)]
    scratch += [pltpu.VMEM((c, tb.shape[1]), F32) for tb in tables]
    scratch += [pltpu.SemaphoreType.DMA]

    def body(*refs):
        pos = 0
        rank_hbm, s_hbm = refs[0], refs[1]
        pos = 2
        if with_q:
            q_hbm = refs[pos]
            pos += 1
        tabs = refs[pos:pos + nt]
        pos += nt
        perm_out, vals_out = refs[pos], refs[pos + 1]
        pos += 2
        if with_q:
            q_out = refs[pos]
            pos += 1
        outs = refs[pos:pos + nt]
        pos += nt
        rank_v, s_v, perm_v, vals_v = refs[pos:pos + 4]
        pos += 4
        if with_q:
            q_v, qn_v = refs[pos:pos + 2]
            pos += 2
        perm_sh, idx_v = refs[pos], refs[pos + 1]
        pos += 2
        bufs = refs[pos:pos + nt]
        sem = refs[pos + nt]

        cid = lax.axis_index("c")
        sid = lax.axis_index("s")

        @pl.when(sid == 0)
        def _():
            pltpu.sync_copy(rank_hbm, rank_v)
            pltpu.sync_copy(s_hbm, s_v)
            if with_q:
                pltpu.sync_copy(q_hbm, q_v)

            def step(cc, carry):
                base = cc * 16
                idx = rank_v[pl.ds(base, 16)]
                msk = idx < k_out
                ids = lax.iota(I32, 16) + base
                plsc.store_scatter(perm_v, [idx], ids, mask=msk)
                plsc.store_scatter(vals_v, [idx], s_v[pl.ds(base, 16)],
                                   mask=msk)
                if with_q:
                    plsc.store_scatter(qn_v, [idx], q_v[pl.ds(base, 16)],
                                       mask=msk)
                return carry

            lax.fori_loop(0, k_in // 16, step, 0)
            pltpu.sync_copy(perm_v, perm_sh)

            @pl.when(cid == 0)
            def _():
                pltpu.sync_copy(perm_v, perm_out)
                pltpu.sync_copy(vals_v, vals_out)
                if with_q:
                    pltpu.sync_copy(qn_v, q_out)

        plsc.subcore_barrier()
        wid = sid * 2 + cid

        @pl.when(wid < n_workers)
        def _():
            base = wid * c
            pltpu.sync_copy(perm_sh.at[pl.ds(base, c)], idx_v)
            for tb, buf, out in zip(tabs, bufs, outs):
                pltpu.async_copy(tb.at[idx_v], buf, sem).wait()
                pltpu.sync_copy(buf, out.at[pl.ds(base, c)])

    fn = pl.kernel(body, out_type=tuple(out_type), mesh=mesh,
                   scratch_types=tuple(scratch),
                   compiler_params=pltpu.CompilerParams(
                       needs_layout_passes=False))
    if with_q:
        return fn(rank_flat, s_flat, q_flat, *tables)
    return fn(rank_flat, s_flat, *tables)


def _readout_body(x_ref, v_ref, mx_ref, sm_ref):
    i = pl.program_id(0)
    xs = x_ref[...] * jnp.tanh(v_ref[...])
    m = jnp.max(xs, axis=0, keepdims=True)
    s = jnp.sum(xs, axis=0, keepdims=True)

    @pl.when(i == 0)
    def _():
        mx_ref[...] = m
        sm_ref[...] = s

    @pl.when(i > 0)
    def _():
        mx_ref[...] = jnp.maximum(mx_ref[...], m)
        sm_ref[...] += s


def _readout(x_gath, vals_col, br):
    n, nhid = x_gath.shape
    return pl.pallas_call(
        _readout_body,
        grid=(n // br,),
        in_specs=[
            pl.BlockSpec((br, nhid), lambda i: (i, 0)),
            pl.BlockSpec((br, 1), lambda i: (i, 0)),
        ],
        out_specs=[
            pl.BlockSpec((1, nhid), lambda i: (0, 0)),
            pl.BlockSpec((1, nhid), lambda i: (0, 0)),
        ],
        out_shape=[
            jax.ShapeDtypeStruct((1, nhid), F32),
            jax.ShapeDtypeStruct((1, nhid), F32),
        ],
    )(x_gath, vals_col)


def kernel(feature, img_info, W_pos, b_pos, g_f, b_f, g_p, b_p,
           W1, b1, W2, b2, W3, b3, Ws1, bs1, Ws2, bs2, Ws3, bs3):
    n = feature.shape[0]
    nf = feature.shape[1]
    nhid = W1.shape[1]
    k1 = math.ceil(0.75 * n)
    k2 = math.ceil(0.75 * k1)
    k3 = math.ceil(0.75 * k2)

    img_p = jnp.pad(img_info, ((0, 0), (0, 128 - img_info.shape[1])))
    wp_p = jnp.pad(W_pos, ((0, 128 - W_pos.shape[0]), (0, 128 - W_pos.shape[1])))
    bp_p = jnp.pad(b_pos, (0, 128 - b_pos.shape[0])).reshape(1, 128)
    feat_p = jnp.pad(feature, ((0, 0), (0, 512 - nf)))
    gf_p = jnp.pad(g_f, (0, 512 - nf)).reshape(1, 512)
    bf_p = jnp.pad(b_f, (0, 512 - nf)).reshape(1, 512)
    gp_p = jnp.pad(g_p, (0, 128 - g_p.shape[0])).reshape(1, 128)
    bpl = jnp.pad(b_p, (0, 128 - b_p.shape[0])).reshape(1, 128)
    w1p = jnp.pad(W1[nf:, :], ((0, 128 - (512 - nf)), (0, 0)))
    ws1_p = jnp.pad(Ws1, ((0, 0), (0, 127)))
    ws2_p = jnp.pad(Ws2, ((0, 0), (0, 127)))
    ws3_p = jnp.pad(Ws3, ((0, 0), (0, 127)))
    bs1_r = jnp.broadcast_to(bs1.reshape(1, 1), (1, 128))
    bs2_r = jnp.broadcast_to(bs2.reshape(1, 1), (1, 128))
    bs3_r = jnp.broadcast_to(bs3.reshape(1, 1), (1, 128))
    b1_r = b1.reshape(1, nhid)
    b2_r = b2.reshape(1, nhid)
    b3_r = b3.reshape(1, nhid)

    lnf, lnp, xw1 = _attn_ln(img_p, wp_p, bp_p, feat_p, gf_p, bf_p, gp_p,
                             bpl, W1, w1p, nhid)
    feat = jnp.concatenate([lnf[:, :nf], lnp[:, :512 - nf]], axis=1)

    d_bf, colmax = _dist_max(feat, 512, 1024)
    t = colmax

    mask1, dinv1 = _mask1_deg(d_bf, t, 1024, 1024)
    x1 = _conv(mask1, xw1, dinv1, b1_r, None, None, 1024, 1024)
    s1 = _score(mask1, x1, dinv1, ws1_p, bs1_r, 1024, 1024)
    rank1 = _rank(s1, s1.reshape(1, n))
    perm1, vals1, featsub2, xg1 = _select_gather(
        rank1.reshape(n), s1.reshape(n), None, k1, [feat, x1])

    q2c = perm1.reshape(k1, 1)
    q2r = perm1.reshape(1, k1)
    mask2, dinv2 = _mask23_deg(featsub2, q2c, q2r, t, 1024, 1024)
    x2, mx1, sm1 = _conv(mask2, xg1, dinv2, b2_r, vals1.reshape(k1, 1), W2,
                         1024, 1024)
    read1 = jnp.concatenate([mx1, sm1 / k1], axis=1)
    s2 = _score(mask2, x2, dinv2, ws2_p, bs2_r, 1024, 1024)
    rank2 = _rank(s2, s2.reshape(1, k1))
    perm2, vals2, q3, featsub3, xg2 = _select_gather(
        rank2.reshape(k1), s2.reshape(k1), perm1, k2, [featsub2, x2])

    q3c = q3.reshape(k2, 1)
    q3r = q3.reshape(1, k2)
    mask3, dinv3 = _mask23_deg(featsub3, q3c, q3r, t, 768, 768)
    x3, mx2, sm2 = _conv(mask3, xg2, dinv3, b3_r, vals2.reshape(k2, 1), W3,
                         768, 768)
    read2 = jnp.concatenate([mx2, sm2 / k2], axis=1)
    s3 = _score(mask3, x3, dinv3, ws3_p, bs3_r, 768, 768)
    rank3 = _rank(s3, s3.reshape(1, k2))
    perm3, vals3, xg3 = _select_gather(
        rank3.reshape(k2), s3.reshape(k2), None, k3, [x3])
    mx3, sm3 = _readout(xg3, vals3.reshape(k3, 1), 192)
    read3 = jnp.concatenate([mx3, sm3 / k3], axis=1)

    return read1 + read2 + read3


# speedup vs baseline: 1.5137x; 1.0285x over previous
import functools
import math

import jax
import jax.numpy as jnp
from jax import lax
from jax.experimental import pallas as pl
from jax.experimental.pallas import tpu as pltpu
from jax.experimental.pallas import tpu_sc as plsc

F32 = jnp.float32
BF16 = jnp.bfloat16
I32 = jnp.int32
LN_EPS = 1e-5


def _dot(a, b, ca, cb):
    return lax.dot_general(a, b, ((ca, cb), ((), ())),
                           preferred_element_type=F32)


def _dotb(a, b, ca, cb):
    return lax.dot_general(a.astype(BF16), b.astype(BF16),
                           ((ca, cb), ((), ())), preferred_element_type=F32)


def _attn_ln_body(img_ref, wp_ref, bp_ref, feat_ref, gf_ref, bf_ref, gp_ref,
                  bpl_ref, w1_ref, w1p_ref, lnf_ref, lnp_ref, xw1_ref,
                  pos_scr):
    i = pl.program_id(0)

    @pl.when(i == 0)
    def _():
        pos_scr[...] = _dot(img_ref[...], wp_ref[...], (1,), (0,)) + bp_ref[...]

    br = feat_ref.shape[0]
    pos_all = pos_scr[...]
    pos_blk = pos_scr[pl.ds(i * br, br), :]
    s = _dotb(pos_blk, pos_all, (1,), (1,)) * (1.0 / math.sqrt(12.0))
    m = jnp.max(s, axis=1, keepdims=True)
    p = jnp.exp(s - m)
    den = jnp.sum(p, axis=1, keepdims=True)
    attn = _dotb(p, pos_all, (1,), (0,)) / den
    mu = jnp.sum(attn, axis=1, keepdims=True) / 12.0
    lane = lax.broadcasted_iota(I32, (br, 128), 1)
    xc = jnp.where(lane < 12, attn - mu, 0.0)
    var = jnp.sum(xc * xc, axis=1, keepdims=True) / 12.0
    lnp = xc / jnp.sqrt(var + LN_EPS) * gp_ref[...] + bpl_ref[...]
    f = feat_ref[...]
    muf = jnp.sum(f, axis=1, keepdims=True) / 500.0
    lane2 = lax.broadcasted_iota(I32, (br, 512), 1)
    xcf = jnp.where(lane2 < 500, f - muf, 0.0)
    varf = jnp.sum(xcf * xcf, axis=1, keepdims=True) / 500.0
    lnf = xcf / jnp.sqrt(varf + LN_EPS) * gf_ref[...] + bf_ref[...]
    lnf_ref[...] = lnf
    lnp_ref[...] = lnp
    xw1_ref[...] = (_dot(lnf, w1_ref[...], (1,), (0,))
                    + _dot(lnp, w1p_ref[...], (1,), (0,)))


def _attn_ln(img_p, wp_p, bp_p, feat_p, gf_p, bf_p, gp_p, bpl, w1, w1p, nhid):
    n = img_p.shape[0]
    return pl.pallas_call(
        _attn_ln_body,
        grid=(n // 256,),
        in_specs=[
            pl.BlockSpec((n, 128), lambda i: (0, 0)),
            pl.BlockSpec((128, 128), lambda i: (0, 0)),
            pl.BlockSpec((1, 128), lambda i: (0, 0)),
            pl.BlockSpec((256, 512), lambda i: (i, 0)),
            pl.BlockSpec((1, 512), lambda i: (0, 0)),
            pl.BlockSpec((1, 512), lambda i: (0, 0)),
            pl.BlockSpec((1, 128), lambda i: (0, 0)),
            pl.BlockSpec((1, 128), lambda i: (0, 0)),
            pl.BlockSpec((512, nhid), lambda i: (0, 0)),
            pl.BlockSpec((128, nhid), lambda i: (0, 0)),
        ],
        out_specs=[
            pl.BlockSpec((256, 512), lambda i: (i, 0)),
            pl.BlockSpec((256, 128), lambda i: (i, 0)),
            pl.BlockSpec((256, nhid), lambda i: (i, 0)),
        ],
        out_shape=[
            jax.ShapeDtypeStruct((n, 512), F32),
            jax.ShapeDtypeStruct((n, 128), F32),
            jax.ShapeDtypeStruct((n, nhid), F32),
        ],
        scratch_shapes=[pltpu.VMEM((n, 128), F32)],
    )(img_p, wp_p, bp_p, feat_p, gf_p, bf_p, gp_p, bpl, w1, w1p)


def _dist_body(fi_ref, fj_ref, d_ref, cm_ref):
    i = pl.program_id(1)
    fi = fi_ref[...]
    fj = fj_ref[...]
    sqi = jnp.sum(fi * fi, axis=1, keepdims=True)
    sqj = _dot(jnp.ones((1, 512), F32), fj * fj, (1,), (1,))
    d = sqi + sqj - 2.0 * _dotb(fi, fj, (1,), (1,))
    d_ref[...] = d.astype(BF16)
    cm = jnp.max(d, axis=0, keepdims=True)

    @pl.when(i == 0)
    def _():
        cm_ref[...] = cm

    @pl.when(i > 0)
    def _():
        cm_ref[...] = jnp.maximum(cm_ref[...], cm)


def _dist_max(feat, bi, bj):
    n = feat.shape[0]
    return pl.pallas_call(
        _dist_body,
        grid=(n // bj, n // bi),
        in_specs=[
            pl.BlockSpec((bi, 512), lambda j, i: (i, 0)),
            pl.BlockSpec((bj, 512), lambda j, i: (j, 0)),
        ],
        out_specs=[
            pl.BlockSpec((bi, bj), lambda j, i: (i, j)),
            pl.BlockSpec((1, bj), lambda j, i: (0, j)),
        ],
        out_shape=[
            jax.ShapeDtypeStruct((n, n), BF16),
            jax.ShapeDtypeStruct((1, n), F32),
        ],
    )(feat, feat)


def _mask1_body(d_ref, t_ref, mask_ref, dinv_ref):
    j = pl.program_id(1)
    i = pl.program_id(0)
    nj = pl.num_programs(1)
    bi, bj = d_ref.shape
    t = 0.5 * jnp.max(t_ref[...])
    gi = lax.broadcasted_iota(I32, (bi, bj), 0) + i * bi
    gj = lax.broadcasted_iota(I32, (bi, bj), 1) + j * bj
    cond = (d_ref[...].astype(F32) < t) & (gj < gi)
    m = jnp.where(cond, 1.0, 0.0).astype(BF16)
    mask_ref[...] = m
    degp = _dot(m, jnp.ones((bj, 128), BF16), (1,), (0,))[:, :1]

    @pl.when(j == 0)
    def _():
        dinv_ref[...] = 1.0 + degp

    @pl.when(j > 0)
    def _():
        dinv_ref[...] += degp

    @pl.when(j == nj - 1)
    def _():
        dinv_ref[...] = 1.0 / jnp.sqrt(dinv_ref[...])


def _mask1_deg(d_bf, t, bi, bj):
    n = d_bf.shape[0]
    return pl.pallas_call(
        _mask1_body,
        grid=(n // bi, n // bj),
        in_specs=[
            pl.BlockSpec((bi, bj), lambda i, j: (i, j)),
            pl.BlockSpec((1, t.shape[1]), lambda i, j: (0, 0)),
        ],
        out_specs=[
            pl.BlockSpec((bi, bj), lambda i, j: (i, j)),
            pl.BlockSpec((bi, 1), lambda i, j: (i, 0)),
        ],
        out_shape=[
            jax.ShapeDtypeStruct((n, n), BF16),
            jax.ShapeDtypeStruct((n, 1), F32),
        ],
    )(d_bf, t)


def _mask23_body(fi_ref, fj_ref, qc_ref, qr_ref, t_ref, mask_ref, dinv_ref):
    j = pl.program_id(1)
    nj = pl.num_programs(1)
    bi = fi_ref.shape[0]
    bj = fj_ref.shape[0]
    fi = fi_ref[...]
    fj = fj_ref[...]
    sqi = jnp.sum(fi * fi, axis=1, keepdims=True)
    sqj = _dot(jnp.ones((1, 512), F32), fj * fj, (1,), (1,))
    d = sqi + sqj - 2.0 * _dotb(fi, fj, (1,), (1,))
    t = 0.5 * jnp.max(t_ref[...])
    cond = (d < t) & (qr_ref[...] < qc_ref[...])
    m = jnp.where(cond, 1.0, 0.0).astype(BF16)
    mask_ref[...] = m
    degp = _dot(m, jnp.ones((bj, 128), BF16), (1,), (0,))[:, :1]

    @pl.when(j == 0)
    def _():
        dinv_ref[...] = 1.0 + degp

    @pl.when(j > 0)
    def _():
        dinv_ref[...] += degp

    @pl.when(j == nj - 1)
    def _():
        dinv_ref[...] = 1.0 / jnp.sqrt(dinv_ref[...])


def _mask23_deg(featsub, qc, qr, t, bi, bj):
    n = featsub.shape[0]
    return pl.pallas_call(
        _mask23_body,
        grid=(n // bi, n // bj),
        in_specs=[
            pl.BlockSpec((bi, 512), lambda i, j: (i, 0)),
            pl.BlockSpec((bj, 512), lambda i, j: (j, 0)),
            pl.BlockSpec((bi, 1), lambda i, j: (i, 0)),
            pl.BlockSpec((1, bj), lambda i, j: (0, j)),
            pl.BlockSpec((1, t.shape[1]), lambda i, j: (0, 0)),
        ],
        out_specs=[
            pl.BlockSpec((bi, bj), lambda i, j: (i, j)),
            pl.BlockSpec((bi, 1), lambda i, j: (i, 0)),
        ],
        out_shape=[
            jax.ShapeDtypeStruct((n, n), BF16),
            jax.ShapeDtypeStruct((n, 1), F32),
        ],
    )(featsub, featsub, qc, qr, t)


def _conv_body(fused, mask_ref, x_ref, dinvj_ref, dinvi_ref, b_ref, *rest):
    if fused:
        v_ref, w_ref, out_ref, mx_ref, sm_ref = rest
    else:
        out_ref, = rest
    i = pl.program_id(0)
    j = pl.program_id(1)
    nj = pl.num_programs(1)
    bi = mask_ref.shape[0]
    bj = mask_ref.shape[1]
    if fused:
        xs = x_ref[...] * jnp.tanh(v_ref[...])
        xw = _dot(xs, w_ref[...], (1,), (0,))

        @pl.when(i == 0)
        def _():
            mro = jnp.max(xs, axis=0, keepdims=True)
            sro = jnp.sum(xs, axis=0, keepdims=True)

            @pl.when(j == 0)
            def _():
                mx_ref[...] = mro
                sm_ref[...] = sro

            @pl.when(j > 0)
            def _():
                mx_ref[...] = jnp.maximum(mx_ref[...], mro)
                sm_ref[...] += sro
    else:
        xw = x_ref[...]
    z = xw * dinvj_ref[...]
    contrib = _dotb(mask_ref[...], z, (1,), (0,))

    @pl.when(j == 0)
    def _():
        out_ref[...] = contrib

    @pl.when(j > 0)
    def _():
        out_ref[...] += contrib

    off = j * bj - i * bi

    @pl.when((off >= 0) & (off < bi))
    def _():
        out_ref[pl.ds(pl.multiple_of(off, bj), bj), :] += z

    @pl.when(j == nj - 1)
    def _():
        out_ref[...] = jnp.maximum(out_ref[...] * dinvi_ref[...] + b_ref[...],
                                   0.0)


def _conv(maskT, xin, dinv, b_row, vals_col, w_next, bi, bj):
    n = maskT.shape[0]
    nhid = xin.shape[1]
    fused = vals_col is not None
    in_specs = [
        pl.BlockSpec((bi, bj), lambda i, j: (i, j)),
        pl.BlockSpec((bj, nhid), lambda i, j: (j, 0)),
        pl.BlockSpec((bj, 1), lambda i, j: (j, 0)),
        pl.BlockSpec((bi, 1), lambda i, j: (i, 0)),
        pl.BlockSpec((1, nhid), lambda i, j: (0, 0)),
    ]
    args = [maskT, xin, dinv, dinv, b_row]
    out_specs = [pl.BlockSpec((bi, nhid), lambda i, j: (i, 0))]
    out_shape = [jax.ShapeDtypeStruct((n, nhid), F32)]
    if fused:
        in_specs += [
            pl.BlockSpec((bj, 1), lambda i, j: (j, 0)),
            pl.BlockSpec((nhid, nhid), lambda i, j: (0, 0)),
        ]
        args += [vals_col, w_next]
        out_specs += [
            pl.BlockSpec((1, nhid), lambda i, j: (0, 0)),
            pl.BlockSpec((1, nhid), lambda i, j: (0, 0)),
        ]
        out_shape += [
            jax.ShapeDtypeStruct((1, nhid), F32),
            jax.ShapeDtypeStruct((1, nhid), F32),
        ]
    res = pl.pallas_call(
        functools.partial(_conv_body, fused),
        grid=(n // bi, n // bj),
        in_specs=in_specs,
        out_specs=out_specs,
        out_shape=out_shape,
    )(*args)
    return res if fused else res[0]


def _score_body(mask_ref, x_ref, dinvj_ref, dinvi_ref, ws_ref, bs_ref,
                out_ref):
    i = pl.program_id(0)
    j = pl.program_id(1)
    nj = pl.num_programs(1)
    bi = mask_ref.shape[0]
    bj = mask_ref.shape[1]
    u = _dot(x_ref[...], ws_ref[...], (1,), (0,)) * dinvj_ref[...]
    contrib = _dotb(mask_ref[...], u, (1,), (0,))

    @pl.when(j == 0)
    def _():
        out_ref[...] = contrib

    @pl.when(j > 0)
    def _():
        out_ref[...] += contrib

    off = j * bj - i * bi

    @pl.when((off >= 0) & (off < bi))
    def _():
        out_ref[pl.ds(pl.multiple_of(off, bj), bj), :] += u

    @pl.when(j == nj - 1)
    def _():
        out_ref[...] = out_ref[...] * dinvi_ref[...] + bs_ref[...]


def _score(maskT, x, dinv, ws_p, bs_row, bi, bj):
    n = maskT.shape[0]
    nhid = x.shape[1]
    out = pl.pallas_call(
        _score_body,
        grid=(n // bi, n // bj),
        in_specs=[
            pl.BlockSpec((bi, bj), lambda i, j: (i, j)),
            pl.BlockSpec((bj, nhid), lambda i, j: (j, 0)),
            pl.BlockSpec((bj, 1), lambda i, j: (j, 0)),
            pl.BlockSpec((bi, 1), lambda i, j: (i, 0)),
            pl.BlockSpec((nhid, 128), lambda i, j: (0, 0)),
            pl.BlockSpec((1, 128), lambda i, j: (0, 0)),
        ],
        out_specs=pl.BlockSpec((bi, 128), lambda i, j: (i, 0)),
        out_shape=jax.ShapeDtypeStruct((n, 128), F32),
    )(maskT, x, dinv, dinv, ws_p, bs_row)
    return out[:, :1]


def _rank_body(sc_ref, sr_ref, rank_ref):
    i = pl.program_id(0)
    bi = sc_ref.shape[0]
    n = sr_ref.shape[1]
    si = sc_ref[...]
    sj = sr_ref[...]
    gi = lax.broadcasted_iota(I32, (bi, n), 0) + i * bi
    gj = lax.broadcasted_iota(I32, (bi, n), 1)
    before = (sj > si) | ((sj == si) & (gj < gi))
    rank_ref[...] = jnp.sum(before.astype(I32), axis=1, keepdims=True)


def _rank(sc, sr):
    n = sc.shape[0]
    bi = 256
    return pl.pallas_call(
        _rank_body,
        grid=(n // bi,),
        in_specs=[
            pl.BlockSpec((bi, 1), lambda i: (i, 0)),
            pl.BlockSpec((1, n), lambda i: (0, 0)),
        ],
        out_specs=pl.BlockSpec((bi, 1), lambda i: (i, 0)),
        out_shape=jax.ShapeDtypeStruct((n, 1), I32),
    )(sc, sr)


def _select_gather(rank_flat, s_flat, q_flat, k_out, tables):
    k_in = rank_flat.shape[0]
    with_q = q_flat is not None
    nt = len(tables)
    n_workers = 32
    c = k_out // n_workers
    if c % 8 != 0:
        c = 64
        n_workers = k_out // c
    assert c * n_workers == k_out and c % 8 == 0 and c <= 128
    mesh = plsc.VectorSubcoreMesh(core_axis_name="c", subcore_axis_name="s")
    out_type = [jax.ShapeDtypeStruct((k_out,), I32),
                jax.ShapeDtypeStruct((k_out,), F32)]
    if with_q:
        out_type.append(jax.ShapeDtypeStruct((k_out,), I32))
    out_type += [jax.ShapeDtypeStruct((k_out, tb.shape[1]), F32)
                 for tb in tables]
    scratch = [pltpu.VMEM((k_in,), I32), pltpu.VMEM((k_in,), F32),
               pltpu.VMEM((k_out,), I32), pltpu.VMEM((k_out,), F32)]
    if with_q:
        scratch += [pltpu.VMEM((k_in,), I32), pltpu.VMEM((k_out,), I32)]
    scratch += [pltpu.VMEM_SHARED((k_out,), I32), pltpu.VMEM((c,), I32)]
    scratch += [pltpu.VMEM((c, tb.shape[1]), F32) for tb in tables]
    scratch += [pltpu.SemaphoreType.DMA]

    def body(*refs):
        pos = 0
        rank_hbm, s_hbm = refs[0], refs[1]
        pos = 2
        if with_q:
            q_hbm = refs[pos]
            pos += 1
        tabs = refs[pos:pos + nt]
        pos += nt
        perm_out, vals_out = refs[pos], refs[pos + 1]
        pos += 2
        if with_q:
            q_out = refs[pos]
            pos += 1
        outs = refs[pos:pos + nt]
        pos += nt
        rank_v, s_v, perm_v, vals_v = refs[pos:pos + 4]
        pos += 4
        if with_q:
            q_v, qn_v = refs[pos:pos + 2]
            pos += 2
        perm_sh, idx_v = refs[pos], refs[pos + 1]
        pos += 2
        bufs = refs[pos:pos + nt]
        sem = refs[pos + nt]

        cid = lax.axis_index("c")
        sid = lax.axis_index("s")

        @pl.when(sid == 0)
        def _():
            pltpu.sync_copy(rank_hbm, rank_v)
            pltpu.sync_copy(s_hbm, s_v)
            if with_q:
                pltpu.sync_copy(q_hbm, q_v)

            def step(cc, carry):
                base = cc * 16
                idx = rank_v[pl.ds(base, 16)]
                msk = idx < k_out
                ids = lax.iota(I32, 16) + base
                plsc.store_scatter(perm_v, [idx], ids, mask=msk)
                plsc.store_scatter(vals_v, [idx], s_v[pl.ds(base, 16)],
                                   mask=msk)
                if with_q:
                    plsc.store_scatter(qn_v, [idx], q_v[pl.ds(base, 16)],
                                       mask=msk)
                return carry

            lax.fori_loop(0, k_in // 16, step, 0)
            pltpu.sync_copy(perm_v, perm_sh)

            @pl.when(cid == 0)
            def _():
                pltpu.sync_copy(perm_v, perm_out)
                pltpu.sync_copy(vals_v, vals_out)
                if with_q:
                    pltpu.sync_copy(qn_v, q_out)

        plsc.subcore_barrier()
        wid = sid * 2 + cid

        @pl.when(wid < n_workers)
        def _():
            base = wid * c
            pltpu.sync_copy(perm_sh.at[pl.ds(base, c)], idx_v)
            for tb, buf, out in zip(tabs, bufs, outs):
                pltpu.async_copy(tb.at[idx_v], buf, sem).wait()
                pltpu.sync_copy(buf, out.at[pl.ds(base, c)])

    fn = pl.kernel(body, out_type=tuple(out_type), mesh=mesh,
                   scratch_types=tuple(scratch),
                   compiler_params=pltpu.CompilerParams(
                       needs_layout_passes=False))
    if with_q:
        return fn(rank_flat, s_flat, q_flat, *tables)
    return fn(rank_flat, s_flat, *tables)


def _readout_body(x_ref, v_ref, mx_ref, sm_ref):
    i = pl.program_id(0)
    xs = x_ref[...] * jnp.tanh(v_ref[...])
    m = jnp.max(xs, axis=0, keepdims=True)
    s = jnp.sum(xs, axis=0, keepdims=True)

    @pl.when(i == 0)
    def _():
        mx_ref[...] = m
        sm_ref[...] = s

    @pl.when(i > 0)
    def _():
        mx_ref[...] = jnp.maximum(mx_ref[...], m)
        sm_ref[...] += s


def _readout(x_gath, vals_col, br):
    n, nhid = x_gath.shape
    return pl.pallas_call(
        _readout_body,
        grid=(n // br,),
        in_specs=[
            pl.BlockSpec((br, nhid), lambda i: (i, 0)),
            pl.BlockSpec((br, 1), lambda i: (i, 0)),
        ],
        out_specs=[
            pl.BlockSpec((1, nhid), lambda i: (0, 0)),
            pl.BlockSpec((1, nhid), lambda i: (0, 0)),
        ],
        out_shape=[
            jax.ShapeDtypeStruct((1, nhid), F32),
            jax.ShapeDtypeStruct((1, nhid), F32),
        ],
    )(x_gath, vals_col)


def kernel(feature, img_info, W_pos, b_pos, g_f, b_f, g_p, b_p,
           W1, b1, W2, b2, W3, b3, Ws1, bs1, Ws2, bs2, Ws3, bs3):
    n = feature.shape[0]
    nf = feature.shape[1]
    nhid = W1.shape[1]
    k1 = math.ceil(0.75 * n)
    k2 = math.ceil(0.75 * k1)
    k3 = math.ceil(0.75 * k2)

    img_p = jnp.pad(img_info, ((0, 0), (0, 128 - img_info.shape[1])))
    wp_p = jnp.pad(W_pos, ((0, 128 - W_pos.shape[0]), (0, 128 - W_pos.shape[1])))
    bp_p = jnp.pad(b_pos, (0, 128 - b_pos.shape[0])).reshape(1, 128)
    feat_p = jnp.pad(feature, ((0, 0), (0, 512 - nf)))
    gf_p = jnp.pad(g_f, (0, 512 - nf)).reshape(1, 512)
    bf_p = jnp.pad(b_f, (0, 512 - nf)).reshape(1, 512)
    gp_p = jnp.pad(g_p, (0, 128 - g_p.shape[0])).reshape(1, 128)
    bpl = jnp.pad(b_p, (0, 128 - b_p.shape[0])).reshape(1, 128)
    w1p = jnp.pad(W1[nf:, :], ((0, 128 - (512 - nf)), (0, 0)))
    ws1_p = jnp.pad(Ws1, ((0, 0), (0, 127)))
    ws2_p = jnp.pad(Ws2, ((0, 0), (0, 127)))
    ws3_p = jnp.pad(Ws3, ((0, 0), (0, 127)))
    bs1_r = jnp.broadcast_to(bs1.reshape(1, 1), (1, 128))
    bs2_r = jnp.broadcast_to(bs2.reshape(1, 1), (1, 128))
    bs3_r = jnp.broadcast_to(bs3.reshape(1, 1), (1, 128))
    b1_r = b1.reshape(1, nhid)
    b2_r = b2.reshape(1, nhid)
    b3_r = b3.reshape(1, nhid)

    lnf, lnp, xw1 = _attn_ln(img_p, wp_p, bp_p, feat_p, gf_p, bf_p, gp_p,
                             bpl, W1, w1p, nhid)
    feat = jnp.concatenate([lnf[:, :nf], lnp[:, :512 - nf]], axis=1)

    d_bf, colmax = _dist_max(feat, 512, 2048)
    t = colmax

    mask1, dinv1 = _mask1_deg(d_bf, t, 1024, 1024)
    x1 = _conv(mask1, xw1, dinv1, b1_r, None, None, 1024, 1024)
    s1 = _score(mask1, x1, dinv1, ws1_p, bs1_r, 1024, 1024)
    rank1 = _rank(s1, s1.reshape(1, n))
    perm1, vals1, featsub2, xg1 = _select_gather(
        rank1.reshape(n), s1.reshape(n), None, k1, [feat, x1])

    q2c = perm1.reshape(k1, 1)
    q2r = perm1.reshape(1, k1)
    mask2, dinv2 = _mask23_deg(featsub2, q2c, q2r, t, 1024, 1024)
    x2, mx1, sm1 = _conv(mask2, xg1, dinv2, b2_r, vals1.reshape(k1, 1), W2,
                         1024, 1024)
    read1 = jnp.concatenate([mx1, sm1 / k1], axis=1)
    s2 = _score(mask2, x2, dinv2, ws2_p, bs2_r, 1024, 1024)
    rank2 = _rank(s2, s2.reshape(1, k1))
    perm2, vals2, q3, featsub3, xg2 = _select_gather(
        rank2.reshape(k1), s2.reshape(k1), perm1, k2, [featsub2, x2])

    q3c = q3.reshape(k2, 1)
    q3r = q3.reshape(1, k2)
    mask3, dinv3 = _mask23_deg(featsub3, q3c, q3r, t, 768, 768)
    x3, mx2, sm2 = _conv(mask3, xg2, dinv3, b3_r, vals2.reshape(k2, 1), W3,
                         768, 768)
    read2 = jnp.concatenate([mx2, sm2 / k2], axis=1)
    s3 = _score(mask3, x3, dinv3, ws3_p, bs3_r, 768, 768)
    rank3 = _rank(s3, s3.reshape(1, k2))
    perm3, vals3, xg3 = _select_gather(
        rank3.reshape(k2), s3.reshape(k2), None, k3, [x3])
    mx3, sm3 = _readout(xg3, vals3.reshape(k3, 1), 192)
    read3 = jnp.concatenate([mx3, sm3 / k3], axis=1)

    return read1 + read2 + read3


# speedup vs baseline: 1.6453x; 1.0870x over previous
import functools
import math

import jax
import jax.numpy as jnp
from jax import lax
from jax.experimental import pallas as pl
from jax.experimental.pallas import tpu as pltpu
from jax.experimental.pallas import tpu_sc as plsc

F32 = jnp.float32
BF16 = jnp.bfloat16
I32 = jnp.int32
LN_EPS = 1e-5


def _dot(a, b, ca, cb):
    return lax.dot_general(a, b, ((ca, cb), ((), ())),
                           preferred_element_type=F32)


def _dotb(a, b, ca, cb):
    return lax.dot_general(a.astype(BF16), b.astype(BF16),
                           ((ca, cb), ((), ())), preferred_element_type=F32)


def _attn_ln_body(img_ref, wp_ref, bp_ref, feat_ref, gf_ref, bf_ref, gp_ref,
                  bpl_ref, w1_ref, w1p_ref, lnf_ref, lnp_ref, xw1_ref,
                  pos_scr):
    i = pl.program_id(0)

    @pl.when(i == 0)
    def _():
        pos_scr[...] = _dot(img_ref[...], wp_ref[...], (1,), (0,)) + bp_ref[...]

    br = feat_ref.shape[0]
    pos_all = pos_scr[...]
    pos_blk = pos_scr[pl.ds(i * br, br), :]
    s = _dotb(pos_blk, pos_all, (1,), (1,)) * (1.0 / math.sqrt(12.0))
    m = jnp.max(s, axis=1, keepdims=True)
    p = jnp.exp(s - m)
    den = jnp.sum(p, axis=1, keepdims=True)
    attn = _dotb(p, pos_all, (1,), (0,)) / den
    mu = jnp.sum(attn, axis=1, keepdims=True) / 12.0
    lane = lax.broadcasted_iota(I32, (br, 128), 1)
    xc = jnp.where(lane < 12, attn - mu, 0.0)
    var = jnp.sum(xc * xc, axis=1, keepdims=True) / 12.0
    lnp = xc / jnp.sqrt(var + LN_EPS) * gp_ref[...] + bpl_ref[...]
    f = feat_ref[...]
    muf = jnp.sum(f, axis=1, keepdims=True) / 500.0
    lane2 = lax.broadcasted_iota(I32, (br, 512), 1)
    xcf = jnp.where(lane2 < 500, f - muf, 0.0)
    varf = jnp.sum(xcf * xcf, axis=1, keepdims=True) / 500.0
    lnf = xcf / jnp.sqrt(varf + LN_EPS) * gf_ref[...] + bf_ref[...]
    lnf_ref[...] = lnf
    lnp_ref[...] = lnp
    xw1_ref[...] = (_dot(lnf, w1_ref[...], (1,), (0,))
                    + _dot(lnp, w1p_ref[...], (1,), (0,)))


def _attn_ln(img_p, wp_p, bp_p, feat_p, gf_p, bf_p, gp_p, bpl, w1, w1p, nhid):
    n = img_p.shape[0]
    return pl.pallas_call(
        _attn_ln_body,
        grid=(n // 256,),
        in_specs=[
            pl.BlockSpec((n, 128), lambda i: (0, 0)),
            pl.BlockSpec((128, 128), lambda i: (0, 0)),
            pl.BlockSpec((1, 128), lambda i: (0, 0)),
            pl.BlockSpec((256, 512), lambda i: (i, 0)),
            pl.BlockSpec((1, 512), lambda i: (0, 0)),
            pl.BlockSpec((1, 512), lambda i: (0, 0)),
            pl.BlockSpec((1, 128), lambda i: (0, 0)),
            pl.BlockSpec((1, 128), lambda i: (0, 0)),
            pl.BlockSpec((512, nhid), lambda i: (0, 0)),
            pl.BlockSpec((128, nhid), lambda i: (0, 0)),
        ],
        out_specs=[
            pl.BlockSpec((256, 512), lambda i: (i, 0)),
            pl.BlockSpec((256, 128), lambda i: (i, 0)),
            pl.BlockSpec((256, nhid), lambda i: (i, 0)),
        ],
        out_shape=[
            jax.ShapeDtypeStruct((n, 512), F32),
            jax.ShapeDtypeStruct((n, 128), F32),
            jax.ShapeDtypeStruct((n, nhid), F32),
        ],
        scratch_shapes=[pltpu.VMEM((n, 128), F32)],
    )(img_p, wp_p, bp_p, feat_p, gf_p, bf_p, gp_p, bpl, w1, w1p)


def _dist_body(fi_ref, fj_ref, d_ref, cm_ref):
    i = pl.program_id(1)
    fi = fi_ref[...]
    fj = fj_ref[...]
    sqi = jnp.sum(fi * fi, axis=1, keepdims=True)
    sqj = _dot(jnp.ones((1, 512), F32), fj * fj, (1,), (1,))
    d = sqi + sqj - 2.0 * _dotb(fi, fj, (1,), (1,))
    d_ref[...] = d.astype(BF16)
    cm = jnp.max(d, axis=0, keepdims=True)

    @pl.when(i == 0)
    def _():
        cm_ref[...] = cm

    @pl.when(i > 0)
    def _():
        cm_ref[...] = jnp.maximum(cm_ref[...], cm)


def _dist_max(feat, bi, bj):
    n = feat.shape[0]
    return pl.pallas_call(
        _dist_body,
        grid=(n // bj, n // bi),
        in_specs=[
            pl.BlockSpec((bi, 512), lambda j, i: (i, 0)),
            pl.BlockSpec((bj, 512), lambda j, i: (j, 0)),
        ],
        out_specs=[
            pl.BlockSpec((bi, bj), lambda j, i: (i, j)),
            pl.BlockSpec((1, bj), lambda j, i: (0, j)),
        ],
        out_shape=[
            jax.ShapeDtypeStruct((n, n), BF16),
            jax.ShapeDtypeStruct((1, n), F32),
        ],
    )(feat, feat)


def _mask1_body(d_ref, t_ref, mask_ref, dinv_ref):
    j = pl.program_id(1)
    i = pl.program_id(0)
    nj = pl.num_programs(1)
    bi, bj = d_ref.shape
    t = 0.5 * jnp.max(t_ref[...])
    gi = lax.broadcasted_iota(I32, (bi, bj), 0) + i * bi
    gj = lax.broadcasted_iota(I32, (bi, bj), 1) + j * bj
    cond = (d_ref[...].astype(F32) < t) & (gj < gi)
    m = jnp.where(cond, 1.0, 0.0).astype(BF16)
    mask_ref[...] = m
    degp = _dot(m, jnp.ones((bj, 128), BF16), (1,), (0,))[:, :1]

    @pl.when(j == 0)
    def _():
        dinv_ref[...] = 1.0 + degp

    @pl.when(j > 0)
    def _():
        dinv_ref[...] += degp

    @pl.when(j == nj - 1)
    def _():
        dinv_ref[...] = 1.0 / jnp.sqrt(dinv_ref[...])


def _mask1_deg(d_bf, t, bi, bj):
    n = d_bf.shape[0]
    return pl.pallas_call(
        _mask1_body,
        grid=(n // bi, n // bj),
        in_specs=[
            pl.BlockSpec((bi, bj), lambda i, j: (i, j)),
            pl.BlockSpec((1, t.shape[1]), lambda i, j: (0, 0)),
        ],
        out_specs=[
            pl.BlockSpec((bi, bj), lambda i, j: (i, j)),
            pl.BlockSpec((bi, 1), lambda i, j: (i, 0)),
        ],
        out_shape=[
            jax.ShapeDtypeStruct((n, n), BF16),
            jax.ShapeDtypeStruct((n, 1), F32),
        ],
    )(d_bf, t)


def _mask23_body(fi_ref, fj_ref, qc_ref, qr_ref, t_ref, mask_ref, dinv_ref):
    j = pl.program_id(1)
    nj = pl.num_programs(1)
    bi = fi_ref.shape[0]
    bj = fj_ref.shape[0]
    fi = fi_ref[...]
    fj = fj_ref[...]
    sqi = jnp.sum(fi * fi, axis=1, keepdims=True)
    sqj = _dot(jnp.ones((1, 512), F32), fj * fj, (1,), (1,))
    d = sqi + sqj - 2.0 * _dotb(fi, fj, (1,), (1,))
    t = 0.5 * jnp.max(t_ref[...])
    cond = (d < t) & (qr_ref[...] < qc_ref[...])
    m = jnp.where(cond, 1.0, 0.0).astype(BF16)
    mask_ref[...] = m
    degp = _dot(m, jnp.ones((bj, 128), BF16), (1,), (0,))[:, :1]

    @pl.when(j == 0)
    def _():
        dinv_ref[...] = 1.0 + degp

    @pl.when(j > 0)
    def _():
        dinv_ref[...] += degp

    @pl.when(j == nj - 1)
    def _():
        dinv_ref[...] = 1.0 / jnp.sqrt(dinv_ref[...])


def _mask23_deg(featsub, qc, qr, t, bi, bj):
    n = featsub.shape[0]
    return pl.pallas_call(
        _mask23_body,
        grid=(n // bi, n // bj),
        in_specs=[
            pl.BlockSpec((bi, 512), lambda i, j: (i, 0)),
            pl.BlockSpec((bj, 512), lambda i, j: (j, 0)),
            pl.BlockSpec((bi, 1), lambda i, j: (i, 0)),
            pl.BlockSpec((1, bj), lambda i, j: (0, j)),
            pl.BlockSpec((1, t.shape[1]), lambda i, j: (0, 0)),
        ],
        out_specs=[
            pl.BlockSpec((bi, bj), lambda i, j: (i, j)),
            pl.BlockSpec((bi, 1), lambda i, j: (i, 0)),
        ],
        out_shape=[
            jax.ShapeDtypeStruct((n, n), BF16),
            jax.ShapeDtypeStruct((n, 1), F32),
        ],
    )(featsub, featsub, qc, qr, t)


def _conv_body(fused, mask_ref, x_ref, dinvj_ref, dinvi_ref, b_ref, *rest):
    if fused:
        v_ref, w_ref, out_ref, mx_ref, sm_ref = rest
    else:
        out_ref, = rest
    i = pl.program_id(0)
    j = pl.program_id(1)
    nj = pl.num_programs(1)
    bi = mask_ref.shape[0]
    bj = mask_ref.shape[1]
    if fused:
        xs = x_ref[...] * jnp.tanh(v_ref[...])
        xw = _dot(xs, w_ref[...], (1,), (0,))

        @pl.when(i == 0)
        def _():
            mro = jnp.max(xs, axis=0, keepdims=True)
            sro = jnp.sum(xs, axis=0, keepdims=True)

            @pl.when(j == 0)
            def _():
                mx_ref[...] = mro
                sm_ref[...] = sro

            @pl.when(j > 0)
            def _():
                mx_ref[...] = jnp.maximum(mx_ref[...], mro)
                sm_ref[...] += sro
    else:
        xw = x_ref[...]
    z = xw * dinvj_ref[...]
    contrib = _dotb(mask_ref[...], z, (1,), (0,))

    @pl.when(j == 0)
    def _():
        out_ref[...] = contrib

    @pl.when(j > 0)
    def _():
        out_ref[...] += contrib

    off = i * bi - j * bj

    @pl.when((off >= 0) & (off < bj))
    def _():
        o = pl.multiple_of(off, bi)
        dv = dinvj_ref[pl.ds(o, bi), :]
        if fused:
            xs_s = x_ref[pl.ds(o, bi), :] * jnp.tanh(v_ref[pl.ds(o, bi), :])
            z_s = _dot(xs_s, w_ref[...], (1,), (0,)) * dv
        else:
            z_s = x_ref[pl.ds(o, bi), :] * dv
        out_ref[...] += z_s

    @pl.when(j == nj - 1)
    def _():
        out_ref[...] = jnp.maximum(out_ref[...] * dinvi_ref[...] + b_ref[...],
                                   0.0)


def _conv(maskT, xin, dinv, b_row, vals_col, w_next, bi, bj):
    n = maskT.shape[0]
    nhid = xin.shape[1]
    fused = vals_col is not None
    in_specs = [
        pl.BlockSpec((bi, bj), lambda i, j: (i, j)),
        pl.BlockSpec((bj, nhid), lambda i, j: (j, 0)),
        pl.BlockSpec((bj, 1), lambda i, j: (j, 0)),
        pl.BlockSpec((bi, 1), lambda i, j: (i, 0)),
        pl.BlockSpec((1, nhid), lambda i, j: (0, 0)),
    ]
    args = [maskT, xin, dinv, dinv, b_row]
    out_specs = [pl.BlockSpec((bi, nhid), lambda i, j: (i, 0))]
    out_shape = [jax.ShapeDtypeStruct((n, nhid), F32)]
    if fused:
        in_specs += [
            pl.BlockSpec((bj, 1), lambda i, j: (j, 0)),
            pl.BlockSpec((nhid, nhid), lambda i, j: (0, 0)),
        ]
        args += [vals_col, w_next]
        out_specs += [
            pl.BlockSpec((1, nhid), lambda i, j: (0, 0)),
            pl.BlockSpec((1, nhid), lambda i, j: (0, 0)),
        ]
        out_shape += [
            jax.ShapeDtypeStruct((1, nhid), F32),
            jax.ShapeDtypeStruct((1, nhid), F32),
        ]
    res = pl.pallas_call(
        functools.partial(_conv_body, fused),
        grid=(n // bi, n // bj),
        in_specs=in_specs,
        out_specs=out_specs,
        out_shape=out_shape,
    )(*args)
    return res if fused else res[0]


def _score_body(mask_ref, x_ref, dinvj_ref, dinvi_ref, ws_ref, bs_ref,
                out_ref):
    i = pl.program_id(0)
    j = pl.program_id(1)
    nj = pl.num_programs(1)
    bi = mask_ref.shape[0]
    bj = mask_ref.shape[1]
    u = _dot(x_ref[...], ws_ref[...], (1,), (0,)) * dinvj_ref[...]
    contrib = _dotb(mask_ref[...], u, (1,), (0,))

    @pl.when(j == 0)
    def _():
        out_ref[...] = contrib

    @pl.when(j > 0)
    def _():
        out_ref[...] += contrib

    off = i * bi - j * bj

    @pl.when((off >= 0) & (off < bj))
    def _():
        o = pl.multiple_of(off, bi)
        u_s = (_dot(x_ref[pl.ds(o, bi), :], ws_ref[...], (1,), (0,))
               * dinvj_ref[pl.ds(o, bi), :])
        out_ref[...] += u_s

    @pl.when(j == nj - 1)
    def _():
        out_ref[...] = out_ref[...] * dinvi_ref[...] + bs_ref[...]


def _score(maskT, x, dinv, ws_p, bs_row, bi, bj):
    n = maskT.shape[0]
    nhid = x.shape[1]
    out = pl.pallas_call(
        _score_body,
        grid=(n // bi, n // bj),
        in_specs=[
            pl.BlockSpec((bi, bj), lambda i, j: (i, j)),
            pl.BlockSpec((bj, nhid), lambda i, j: (j, 0)),
            pl.BlockSpec((bj, 1), lambda i, j: (j, 0)),
            pl.BlockSpec((bi, 1), lambda i, j: (i, 0)),
            pl.BlockSpec((nhid, 128), lambda i, j: (0, 0)),
            pl.BlockSpec((1, 128), lambda i, j: (0, 0)),
        ],
        out_specs=pl.BlockSpec((bi, 128), lambda i, j: (i, 0)),
        out_shape=jax.ShapeDtypeStruct((n, 128), F32),
    )(maskT, x, dinv, dinv, ws_p, bs_row)
    return out[:, :1]


def _rank_body(sc_ref, sr_ref, rank_ref):
    i = pl.program_id(0)
    bi = sc_ref.shape[0]
    n = sr_ref.shape[1]
    si = sc_ref[...]
    sj = sr_ref[...]
    gi = lax.broadcasted_iota(I32, (bi, n), 0) + i * bi
    gj = lax.broadcasted_iota(I32, (bi, n), 1)
    before = (sj > si) | ((sj == si) & (gj < gi))
    rank_ref[...] = jnp.sum(before.astype(I32), axis=1, keepdims=True)


def _rank(sc, sr):
    n = sc.shape[0]
    bi = 256
    return pl.pallas_call(
        _rank_body,
        grid=(n // bi,),
        in_specs=[
            pl.BlockSpec((bi, 1), lambda i: (i, 0)),
            pl.BlockSpec((1, n), lambda i: (0, 0)),
        ],
        out_specs=pl.BlockSpec((bi, 1), lambda i: (i, 0)),
        out_shape=jax.ShapeDtypeStruct((n, 1), I32),
    )(sc, sr)


def _select_gather(rank_flat, s_flat, q_flat, k_out, tables):
    k_in = rank_flat.shape[0]
    with_q = q_flat is not None
    nt = len(tables)
    n_workers = 32
    c = k_out // n_workers
    if c % 8 != 0:
        c = 64
        n_workers = k_out // c
    assert c * n_workers == k_out and c % 8 == 0 and c <= 128
    mesh = plsc.VectorSubcoreMesh(core_axis_name="c", subcore_axis_name="s")
    out_type = [jax.ShapeDtypeStruct((k_out,), I32),
                jax.ShapeDtypeStruct((k_out,), F32)]
    if with_q:
        out_type.append(jax.ShapeDtypeStruct((k_out,), I32))
    out_type += [jax.ShapeDtypeStruct((k_out, tb.shape[1]), F32)
                 for tb in tables]
    scratch = [pltpu.VMEM((k_in,), I32), pltpu.VMEM((k_in,), F32),
               pltpu.VMEM((k_out,), I32), pltpu.VMEM((k_out,), F32)]
    if with_q:
        scratch += [pltpu.VMEM((k_in,), I32), pltpu.VMEM((k_out,), I32)]
    scratch += [pltpu.VMEM_SHARED((k_out,), I32), pltpu.VMEM((c,), I32)]
    scratch += [pltpu.VMEM((c, tb.shape[1]), F32) for tb in tables]
    scratch += [pltpu.SemaphoreType.DMA]

    def body(*refs):
        pos = 0
        rank_hbm, s_hbm = refs[0], refs[1]
        pos = 2
        if with_q:
            q_hbm = refs[pos]
            pos += 1
        tabs = refs[pos:pos + nt]
        pos += nt
        perm_out, vals_out = refs[pos], refs[pos + 1]
        pos += 2
        if with_q:
            q_out = refs[pos]
            pos += 1
        outs = refs[pos:pos + nt]
        pos += nt
        rank_v, s_v, perm_v, vals_v = refs[pos:pos + 4]
        pos += 4
        if with_q:
            q_v, qn_v = refs[pos:pos + 2]
            pos += 2
        perm_sh, idx_v = refs[pos], refs[pos + 1]
        pos += 2
        bufs = refs[pos:pos + nt]
        sem = refs[pos + nt]

        cid = lax.axis_index("c")
        sid = lax.axis_index("s")

        @pl.when(sid == 0)
        def _():
            pltpu.sync_copy(rank_hbm, rank_v)
            pltpu.sync_copy(s_hbm, s_v)
            if with_q:
                pltpu.sync_copy(q_hbm, q_v)

            def step(cc, carry):
                base = cc * 16
                idx = rank_v[pl.ds(base, 16)]
                msk = idx < k_out
                ids = lax.iota(I32, 16) + base
                plsc.store_scatter(perm_v, [idx], ids, mask=msk)
                plsc.store_scatter(vals_v, [idx], s_v[pl.ds(base, 16)],
                                   mask=msk)
                if with_q:
                    plsc.store_scatter(qn_v, [idx], q_v[pl.ds(base, 16)],
                                       mask=msk)
                return carry

            lax.fori_loop(0, k_in // 16, step, 0)
            pltpu.sync_copy(perm_v, perm_sh)

            @pl.when(cid == 0)
            def _():
                pltpu.sync_copy(perm_v, perm_out)
                pltpu.sync_copy(vals_v, vals_out)
                if with_q:
                    pltpu.sync_copy(qn_v, q_out)

        plsc.subcore_barrier()
        wid = sid * 2 + cid

        @pl.when(wid < n_workers)
        def _():
            base = wid * c
            pltpu.sync_copy(perm_sh.at[pl.ds(base, c)], idx_v)
            for tb, buf, out in zip(tabs, bufs, outs):
                pltpu.async_copy(tb.at[idx_v], buf, sem).wait()
                pltpu.sync_copy(buf, out.at[pl.ds(base, c)])

    fn = pl.kernel(body, out_type=tuple(out_type), mesh=mesh,
                   scratch_types=tuple(scratch),
                   compiler_params=pltpu.CompilerParams(
                       needs_layout_passes=False))
    if with_q:
        return fn(rank_flat, s_flat, q_flat, *tables)
    return fn(rank_flat, s_flat, *tables)


def _readout_body(x_ref, v_ref, mx_ref, sm_ref):
    i = pl.program_id(0)
    xs = x_ref[...] * jnp.tanh(v_ref[...])
    m = jnp.max(xs, axis=0, keepdims=True)
    s = jnp.sum(xs, axis=0, keepdims=True)

    @pl.when(i == 0)
    def _():
        mx_ref[...] = m
        sm_ref[...] = s

    @pl.when(i > 0)
    def _():
        mx_ref[...] = jnp.maximum(mx_ref[...], m)
        sm_ref[...] += s


def _readout(x_gath, vals_col, br):
    n, nhid = x_gath.shape
    return pl.pallas_call(
        _readout_body,
        grid=(n // br,),
        in_specs=[
            pl.BlockSpec((br, nhid), lambda i: (i, 0)),
            pl.BlockSpec((br, 1), lambda i: (i, 0)),
        ],
        out_specs=[
            pl.BlockSpec((1, nhid), lambda i: (0, 0)),
            pl.BlockSpec((1, nhid), lambda i: (0, 0)),
        ],
        out_shape=[
            jax.ShapeDtypeStruct((1, nhid), F32),
            jax.ShapeDtypeStruct((1, nhid), F32),
        ],
    )(x_gath, vals_col)


def kernel(feature, img_info, W_pos, b_pos, g_f, b_f, g_p, b_p,
           W1, b1, W2, b2, W3, b3, Ws1, bs1, Ws2, bs2, Ws3, bs3):
    n = feature.shape[0]
    nf = feature.shape[1]
    nhid = W1.shape[1]
    k1 = math.ceil(0.75 * n)
    k2 = math.ceil(0.75 * k1)
    k3 = math.ceil(0.75 * k2)

    img_p = jnp.pad(img_info, ((0, 0), (0, 128 - img_info.shape[1])))
    wp_p = jnp.pad(W_pos, ((0, 128 - W_pos.shape[0]), (0, 128 - W_pos.shape[1])))
    bp_p = jnp.pad(b_pos, (0, 128 - b_pos.shape[0])).reshape(1, 128)
    feat_p = jnp.pad(feature, ((0, 0), (0, 512 - nf)))
    gf_p = jnp.pad(g_f, (0, 512 - nf)).reshape(1, 512)
    bf_p = jnp.pad(b_f, (0, 512 - nf)).reshape(1, 512)
    gp_p = jnp.pad(g_p, (0, 128 - g_p.shape[0])).reshape(1, 128)
    bpl = jnp.pad(b_p, (0, 128 - b_p.shape[0])).reshape(1, 128)
    w1p = jnp.pad(W1[nf:, :], ((0, 128 - (512 - nf)), (0, 0)))
    ws1_p = jnp.pad(Ws1, ((0, 0), (0, 127)))
    ws2_p = jnp.pad(Ws2, ((0, 0), (0, 127)))
    ws3_p = jnp.pad(Ws3, ((0, 0), (0, 127)))
    bs1_r = jnp.broadcast_to(bs1.reshape(1, 1), (1, 128))
    bs2_r = jnp.broadcast_to(bs2.reshape(1, 1), (1, 128))
    bs3_r = jnp.broadcast_to(bs3.reshape(1, 1), (1, 128))
    b1_r = b1.reshape(1, nhid)
    b2_r = b2.reshape(1, nhid)
    b3_r = b3.reshape(1, nhid)

    lnf, lnp, xw1 = _attn_ln(img_p, wp_p, bp_p, feat_p, gf_p, bf_p, gp_p,
                             bpl, W1, w1p, nhid)
    feat = jnp.concatenate([lnf[:, :nf], lnp[:, :512 - nf]], axis=1)

    d_bf, colmax = _dist_max(feat, 512, 2048)
    t = colmax

    mask1, dinv1 = _mask1_deg(d_bf, t, 1024, 2048)
    x1 = _conv(mask1, xw1, dinv1, b1_r, None, None, 1024, 4096)
    s1 = _score(mask1, x1, dinv1, ws1_p, bs1_r, 1024, 4096)
    rank1 = _rank(s1, s1.reshape(1, n))
    perm1, vals1, featsub2, xg1 = _select_gather(
        rank1.reshape(n), s1.reshape(n), None, k1, [feat, x1])

    q2c = perm1.reshape(k1, 1)
    q2r = perm1.reshape(1, k1)
    mask2, dinv2 = _mask23_deg(featsub2, q2c, q2r, t, 1024, 1536)
    x2, mx1, sm1 = _conv(mask2, xg1, dinv2, b2_r, vals1.reshape(k1, 1), W2,
                         1024, 3072)
    read1 = jnp.concatenate([mx1, sm1 / k1], axis=1)
    s2 = _score(mask2, x2, dinv2, ws2_p, bs2_r, 1024, 3072)
    rank2 = _rank(s2, s2.reshape(1, k1))
    perm2, vals2, q3, featsub3, xg2 = _select_gather(
        rank2.reshape(k1), s2.reshape(k1), perm1, k2, [featsub2, x2])

    q3c = q3.reshape(k2, 1)
    q3r = q3.reshape(1, k2)
    mask3, dinv3 = _mask23_deg(featsub3, q3c, q3r, t, 768, 1152)
    x3, mx2, sm2 = _conv(mask3, xg2, dinv3, b3_r, vals2.reshape(k2, 1), W3,
                         768, 2304)
    read2 = jnp.concatenate([mx2, sm2 / k2], axis=1)
    s3 = _score(mask3, x3, dinv3, ws3_p, bs3_r, 768, 2304)
    rank3 = _rank(s3, s3.reshape(1, k2))
    perm3, vals3, xg3 = _select_gather(
        rank3.reshape(k2), s3.reshape(k2), None, k3, [x3])
    mx3, sm3 = _readout(xg3, vals3.reshape(k3, 1), 192)
    read3 = jnp.concatenate([mx3, sm3 / k3], axis=1)

    return read1 + read2 + read3


# speedup vs baseline: 1.6505x; 1.0031x over previous
import functools
import math

import jax
import jax.numpy as jnp
from jax import lax
from jax.experimental import pallas as pl
from jax.experimental.pallas import tpu as pltpu
from jax.experimental.pallas import tpu_sc as plsc

F32 = jnp.float32
BF16 = jnp.bfloat16
I32 = jnp.int32
LN_EPS = 1e-5


def _dot(a, b, ca, cb):
    return lax.dot_general(a, b, ((ca, cb), ((), ())),
                           preferred_element_type=F32)


def _dotb(a, b, ca, cb):
    return lax.dot_general(a.astype(BF16), b.astype(BF16),
                           ((ca, cb), ((), ())), preferred_element_type=F32)


def _attn_ln_body(img_ref, wp_ref, bp_ref, feat_ref, gf_ref, bf_ref, gp_ref,
                  bpl_ref, w1_ref, w1p_ref, lnf_ref, lnp_ref, xw1_ref,
                  pos_scr):
    i = pl.program_id(0)

    @pl.when(i == 0)
    def _():
        pos_scr[...] = _dot(img_ref[...], wp_ref[...], (1,), (0,)) + bp_ref[...]

    br = feat_ref.shape[0]
    pos_all = pos_scr[...]
    pos_blk = pos_scr[pl.ds(i * br, br), :]
    s = _dotb(pos_blk, pos_all, (1,), (1,)) * (1.0 / math.sqrt(12.0))
    m = jnp.max(s, axis=1, keepdims=True)
    p = jnp.exp(s - m)
    den = jnp.sum(p, axis=1, keepdims=True)
    attn = _dotb(p, pos_all, (1,), (0,)) / den
    mu = jnp.sum(attn, axis=1, keepdims=True) / 12.0
    lane = lax.broadcasted_iota(I32, (br, 128), 1)
    xc = jnp.where(lane < 12, attn - mu, 0.0)
    var = jnp.sum(xc * xc, axis=1, keepdims=True) / 12.0
    lnp = xc / jnp.sqrt(var + LN_EPS) * gp_ref[...] + bpl_ref[...]
    f = feat_ref[...]
    muf = jnp.sum(f, axis=1, keepdims=True) / 500.0
    lane2 = lax.broadcasted_iota(I32, (br, 512), 1)
    xcf = jnp.where(lane2 < 500, f - muf, 0.0)
    varf = jnp.sum(xcf * xcf, axis=1, keepdims=True) / 500.0
    lnf = xcf / jnp.sqrt(varf + LN_EPS) * gf_ref[...] + bf_ref[...]
    lnf_ref[...] = lnf
    lnp_ref[...] = lnp
    xw1_ref[...] = (_dot(lnf, w1_ref[...], (1,), (0,))
                    + _dot(lnp, w1p_ref[...], (1,), (0,)))


def _attn_ln(img_p, wp_p, bp_p, feat_p, gf_p, bf_p, gp_p, bpl, w1, w1p, nhid):
    n = img_p.shape[0]
    return pl.pallas_call(
        _attn_ln_body,
        grid=(n // 256,),
        in_specs=[
            pl.BlockSpec((n, 128), lambda i: (0, 0)),
            pl.BlockSpec((128, 128), lambda i: (0, 0)),
            pl.BlockSpec((1, 128), lambda i: (0, 0)),
            pl.BlockSpec((256, 512), lambda i: (i, 0)),
            pl.BlockSpec((1, 512), lambda i: (0, 0)),
            pl.BlockSpec((1, 512), lambda i: (0, 0)),
            pl.BlockSpec((1, 128), lambda i: (0, 0)),
            pl.BlockSpec((1, 128), lambda i: (0, 0)),
            pl.BlockSpec((512, nhid), lambda i: (0, 0)),
            pl.BlockSpec((128, nhid), lambda i: (0, 0)),
        ],
        out_specs=[
            pl.BlockSpec((256, 512), lambda i: (i, 0)),
            pl.BlockSpec((256, 128), lambda i: (i, 0)),
            pl.BlockSpec((256, nhid), lambda i: (i, 0)),
        ],
        out_shape=[
            jax.ShapeDtypeStruct((n, 512), F32),
            jax.ShapeDtypeStruct((n, 128), F32),
            jax.ShapeDtypeStruct((n, nhid), F32),
        ],
        scratch_shapes=[pltpu.VMEM((n, 128), F32)],
    )(img_p, wp_p, bp_p, feat_p, gf_p, bf_p, gp_p, bpl, w1, w1p)


def _dist_body(fi_ref, fj_ref, d_ref, cm_ref):
    i = pl.program_id(1)
    fi = fi_ref[...]
    fj = fj_ref[...]
    sqi = jnp.sum(fi * fi, axis=1, keepdims=True)
    sqj = _dot(jnp.ones((1, 512), F32), fj * fj, (1,), (1,))
    d = sqi + sqj - 2.0 * _dotb(fi, fj, (1,), (1,))
    d_ref[...] = d.astype(BF16)
    cm = jnp.max(d, axis=0, keepdims=True)

    @pl.when(i == 0)
    def _():
        cm_ref[...] = cm

    @pl.when(i > 0)
    def _():
        cm_ref[...] = jnp.maximum(cm_ref[...], cm)


def _dist_max(feat, bi, bj):
    n = feat.shape[0]
    return pl.pallas_call(
        _dist_body,
        grid=(n // bj, n // bi),
        in_specs=[
            pl.BlockSpec((bi, 512), lambda j, i: (i, 0)),
            pl.BlockSpec((bj, 512), lambda j, i: (j, 0)),
        ],
        out_specs=[
            pl.BlockSpec((bi, bj), lambda j, i: (i, j)),
            pl.BlockSpec((1, bj), lambda j, i: (0, j)),
        ],
        out_shape=[
            jax.ShapeDtypeStruct((n, n), BF16),
            jax.ShapeDtypeStruct((1, n), F32),
        ],
    )(feat, feat)


def _mask1_body(d_ref, t_ref, mask_ref, dinv_ref):
    j = pl.program_id(1)
    i = pl.program_id(0)
    nj = pl.num_programs(1)
    bi, bj = d_ref.shape
    t = 0.5 * jnp.max(t_ref[...])
    gi = lax.broadcasted_iota(I32, (bi, bj), 0) + i * bi
    gj = lax.broadcasted_iota(I32, (bi, bj), 1) + j * bj
    cond = (d_ref[...].astype(F32) < t) & (gj < gi)
    m = jnp.where(cond, 1.0, 0.0).astype(BF16)
    mask_ref[...] = m
    degp = _dot(m, jnp.ones((bj, 128), BF16), (1,), (0,))[:, :1]

    @pl.when(j == 0)
    def _():
        dinv_ref[...] = 1.0 + degp

    @pl.when(j > 0)
    def _():
        dinv_ref[...] += degp

    @pl.when(j == nj - 1)
    def _():
        dinv_ref[...] = 1.0 / jnp.sqrt(dinv_ref[...])


def _mask1_deg(d_bf, t, bi, bj):
    n = d_bf.shape[0]
    return pl.pallas_call(
        _mask1_body,
        grid=(n // bi, n // bj),
        in_specs=[
            pl.BlockSpec((bi, bj), lambda i, j: (i, j)),
            pl.BlockSpec((1, t.shape[1]), lambda i, j: (0, 0)),
        ],
        out_specs=[
            pl.BlockSpec((bi, bj), lambda i, j: (i, j)),
            pl.BlockSpec((bi, 1), lambda i, j: (i, 0)),
        ],
        out_shape=[
            jax.ShapeDtypeStruct((n, n), BF16),
            jax.ShapeDtypeStruct((n, 1), F32),
        ],
    )(d_bf, t)


def _mask23_body(fi_ref, fj_ref, qc_ref, qr_ref, t_ref, mask_ref, dinv_ref):
    j = pl.program_id(1)
    nj = pl.num_programs(1)
    bi = fi_ref.shape[0]
    bj = fj_ref.shape[0]
    fi = fi_ref[...]
    fj = fj_ref[...]
    sqi = jnp.sum(fi * fi, axis=1, keepdims=True)
    sqj = _dot(jnp.ones((1, 512), F32), fj * fj, (1,), (1,))
    d = sqi + sqj - 2.0 * _dotb(fi, fj, (1,), (1,))
    t = 0.5 * jnp.max(t_ref[...])
    cond = (d < t) & (qr_ref[...] < qc_ref[...])
    m = jnp.where(cond, 1.0, 0.0).astype(BF16)
    mask_ref[...] = m
    degp = _dot(m, jnp.ones((bj, 128), BF16), (1,), (0,))[:, :1]

    @pl.when(j == 0)
    def _():
        dinv_ref[...] = 1.0 + degp

    @pl.when(j > 0)
    def _():
        dinv_ref[...] += degp

    @pl.when(j == nj - 1)
    def _():
        dinv_ref[...] = 1.0 / jnp.sqrt(dinv_ref[...])


def _mask23_deg(featsub, qc, qr, t, bi, bj):
    n = featsub.shape[0]
    return pl.pallas_call(
        _mask23_body,
        grid=(n // bi, n // bj),
        in_specs=[
            pl.BlockSpec((bi, 512), lambda i, j: (i, 0)),
            pl.BlockSpec((bj, 512), lambda i, j: (j, 0)),
            pl.BlockSpec((bi, 1), lambda i, j: (i, 0)),
            pl.BlockSpec((1, bj), lambda i, j: (0, j)),
            pl.BlockSpec((1, t.shape[1]), lambda i, j: (0, 0)),
        ],
        out_specs=[
            pl.BlockSpec((bi, bj), lambda i, j: (i, j)),
            pl.BlockSpec((bi, 1), lambda i, j: (i, 0)),
        ],
        out_shape=[
            jax.ShapeDtypeStruct((n, n), BF16),
            jax.ShapeDtypeStruct((n, 1), F32),
        ],
    )(featsub, featsub, qc, qr, t)


def _conv_body(fused, mask_ref, x_ref, dinvj_ref, dinvi_ref, b_ref, *rest):
    if fused:
        v_ref, w_ref, out_ref, mx_ref, sm_ref = rest
    else:
        out_ref, = rest
    i = pl.program_id(0)
    j = pl.program_id(1)
    nj = pl.num_programs(1)
    bi = mask_ref.shape[0]
    bj = mask_ref.shape[1]
    if fused:
        xs = x_ref[...] * jnp.tanh(v_ref[...])
        xw = _dot(xs, w_ref[...], (1,), (0,))

        @pl.when(i == 0)
        def _():
            mro = jnp.max(xs, axis=0, keepdims=True)
            sro = jnp.sum(xs, axis=0, keepdims=True)

            @pl.when(j == 0)
            def _():
                mx_ref[...] = mro
                sm_ref[...] = sro

            @pl.when(j > 0)
            def _():
                mx_ref[...] = jnp.maximum(mx_ref[...], mro)
                sm_ref[...] += sro
    else:
        xw = x_ref[...]
    z = xw * dinvj_ref[...]
    contrib = _dotb(mask_ref[...], z, (1,), (0,))

    @pl.when(j == 0)
    def _():
        out_ref[...] = contrib

    @pl.when(j > 0)
    def _():
        out_ref[...] += contrib

    off = i * bi - j * bj

    @pl.when((off >= 0) & (off < bj))
    def _():
        o = pl.multiple_of(off, bi)
        dv = dinvj_ref[pl.ds(o, bi), :]
        if fused:
            xs_s = x_ref[pl.ds(o, bi), :] * jnp.tanh(v_ref[pl.ds(o, bi), :])
            z_s = _dot(xs_s, w_ref[...], (1,), (0,)) * dv
        else:
            z_s = x_ref[pl.ds(o, bi), :] * dv
        out_ref[...] += z_s

    @pl.when(j == nj - 1)
    def _():
        out_ref[...] = jnp.maximum(out_ref[...] * dinvi_ref[...] + b_ref[...],
                                   0.0)


def _conv(maskT, xin, dinv, b_row, vals_col, w_next, bi, bj):
    n = maskT.shape[0]
    nhid = xin.shape[1]
    fused = vals_col is not None
    in_specs = [
        pl.BlockSpec((bi, bj), lambda i, j: (i, j)),
        pl.BlockSpec((bj, nhid), lambda i, j: (j, 0)),
        pl.BlockSpec((bj, 1), lambda i, j: (j, 0)),
        pl.BlockSpec((bi, 1), lambda i, j: (i, 0)),
        pl.BlockSpec((1, nhid), lambda i, j: (0, 0)),
    ]
    args = [maskT, xin, dinv, dinv, b_row]
    out_specs = [pl.BlockSpec((bi, nhid), lambda i, j: (i, 0))]
    out_shape = [jax.ShapeDtypeStruct((n, nhid), F32)]
    if fused:
        in_specs += [
            pl.BlockSpec((bj, 1), lambda i, j: (j, 0)),
            pl.BlockSpec((nhid, nhid), lambda i, j: (0, 0)),
        ]
        args += [vals_col, w_next]
        out_specs += [
            pl.BlockSpec((1, nhid), lambda i, j: (0, 0)),
            pl.BlockSpec((1, nhid), lambda i, j: (0, 0)),
        ]
        out_shape += [
            jax.ShapeDtypeStruct((1, nhid), F32),
            jax.ShapeDtypeStruct((1, nhid), F32),
        ]
    res = pl.pallas_call(
        functools.partial(_conv_body, fused),
        grid=(n // bi, n // bj),
        in_specs=in_specs,
        out_specs=out_specs,
        out_shape=out_shape,
    )(*args)
    return res if fused else res[0]


def _score_body(mask_ref, x_ref, dinvj_ref, dinvi_ref, ws_ref, bs_ref,
                out_ref):
    i = pl.program_id(0)
    j = pl.program_id(1)
    nj = pl.num_programs(1)
    bi = mask_ref.shape[0]
    bj = mask_ref.shape[1]
    u = _dot(x_ref[...], ws_ref[...], (1,), (0,)) * dinvj_ref[...]
    contrib = _dotb(mask_ref[...], u, (1,), (0,))

    @pl.when(j == 0)
    def _():
        out_ref[...] = contrib

    @pl.when(j > 0)
    def _():
        out_ref[...] += contrib

    off = i * bi - j * bj

    @pl.when((off >= 0) & (off < bj))
    def _():
        o = pl.multiple_of(off, bi)
        u_s = (_dot(x_ref[pl.ds(o, bi), :], ws_ref[...], (1,), (0,))
               * dinvj_ref[pl.ds(o, bi), :])
        out_ref[...] += u_s

    @pl.when(j == nj - 1)
    def _():
        out_ref[...] = out_ref[...] * dinvi_ref[...] + bs_ref[...]


def _score(maskT, x, dinv, ws_p, bs_row, bi, bj):
    n = maskT.shape[0]
    nhid = x.shape[1]
    out = pl.pallas_call(
        _score_body,
        grid=(n // bi, n // bj),
        in_specs=[
            pl.BlockSpec((bi, bj), lambda i, j: (i, j)),
            pl.BlockSpec((bj, nhid), lambda i, j: (j, 0)),
            pl.BlockSpec((bj, 1), lambda i, j: (j, 0)),
            pl.BlockSpec((bi, 1), lambda i, j: (i, 0)),
            pl.BlockSpec((nhid, 128), lambda i, j: (0, 0)),
            pl.BlockSpec((1, 128), lambda i, j: (0, 0)),
        ],
        out_specs=pl.BlockSpec((bi, 128), lambda i, j: (i, 0)),
        out_shape=jax.ShapeDtypeStruct((n, 128), F32),
    )(maskT, x, dinv, dinv, ws_p, bs_row)
    return out[:, :1]


def _rank_body(sc_ref, sr_ref, rank_ref):
    i = pl.program_id(0)
    bi = sc_ref.shape[0]
    n = sr_ref.shape[1]
    si = sc_ref[...]
    sj = sr_ref[...]
    gi = lax.broadcasted_iota(I32, (bi, n), 0) + i * bi
    gj = lax.broadcasted_iota(I32, (bi, n), 1)
    before = (sj > si) | ((sj == si) & (gj < gi))
    rank_ref[...] = jnp.sum(before.astype(I32), axis=1, keepdims=True)


def _rank(sc, sr):
    n = sc.shape[0]
    bi = 256
    return pl.pallas_call(
        _rank_body,
        grid=(n // bi,),
        in_specs=[
            pl.BlockSpec((bi, 1), lambda i: (i, 0)),
            pl.BlockSpec((1, n), lambda i: (0, 0)),
        ],
        out_specs=pl.BlockSpec((bi, 1), lambda i: (i, 0)),
        out_shape=jax.ShapeDtypeStruct((n, 1), I32),
    )(sc, sr)


def _select_gather(rank_flat, s_flat, q_flat, k_out, tables):
    k_in = rank_flat.shape[0]
    with_q = q_flat is not None
    nt = len(tables)
    n_workers = 32
    c = k_out // n_workers
    if c % 8 != 0:
        c = 64
        n_workers = k_out // c
    assert c * n_workers == k_out and c % 8 == 0 and c <= 128
    mesh = plsc.VectorSubcoreMesh(core_axis_name="c", subcore_axis_name="s")
    out_type = [jax.ShapeDtypeStruct((k_out,), I32),
                jax.ShapeDtypeStruct((k_out,), F32)]
    if with_q:
        out_type.append(jax.ShapeDtypeStruct((k_out,), I32))
    out_type += [jax.ShapeDtypeStruct((k_out, tb.shape[1]), F32)
                 for tb in tables]
    scratch = [pltpu.VMEM((k_in,), I32), pltpu.VMEM((k_in,), F32),
               pltpu.VMEM((k_out,), I32), pltpu.VMEM((k_out,), F32)]
    if with_q:
        scratch += [pltpu.VMEM((k_in,), I32), pltpu.VMEM((k_out,), I32)]
    scratch += [pltpu.VMEM_SHARED((k_out,), I32), pltpu.VMEM((c,), I32)]
    scratch += [pltpu.VMEM((c, tb.shape[1]), F32) for tb in tables]
    scratch += [pltpu.SemaphoreType.DMA]

    def body(*refs):
        pos = 0
        rank_hbm, s_hbm = refs[0], refs[1]
        pos = 2
        if with_q:
            q_hbm = refs[pos]
            pos += 1
        tabs = refs[pos:pos + nt]
        pos += nt
        perm_out, vals_out = refs[pos], refs[pos + 1]
        pos += 2
        if with_q:
            q_out = refs[pos]
            pos += 1
        outs = refs[pos:pos + nt]
        pos += nt
        rank_v, s_v, perm_v, vals_v = refs[pos:pos + 4]
        pos += 4
        if with_q:
            q_v, qn_v = refs[pos:pos + 2]
            pos += 2
        perm_sh, idx_v = refs[pos], refs[pos + 1]
        pos += 2
        bufs = refs[pos:pos + nt]
        sem = refs[pos + nt]

        cid = lax.axis_index("c")
        sid = lax.axis_index("s")

        @pl.when(sid == 0)
        def _():
            pltpu.sync_copy(rank_hbm, rank_v)
            pltpu.sync_copy(s_hbm, s_v)
            if with_q:
                pltpu.sync_copy(q_hbm, q_v)

            def step(cc, carry):
                base = cc * 16
                idx = rank_v[pl.ds(base, 16)]
                msk = idx < k_out
                ids = lax.iota(I32, 16) + base
                plsc.store_scatter(perm_v, [idx], ids, mask=msk)
                plsc.store_scatter(vals_v, [idx], s_v[pl.ds(base, 16)],
                                   mask=msk)
                if with_q:
                    plsc.store_scatter(qn_v, [idx], q_v[pl.ds(base, 16)],
                                       mask=msk)
                return carry

            lax.fori_loop(0, k_in // 16, step, 0)
            pltpu.sync_copy(perm_v, perm_sh)

            @pl.when(cid == 0)
            def _():
                pltpu.sync_copy(perm_v, perm_out)
                pltpu.sync_copy(vals_v, vals_out)
                if with_q:
                    pltpu.sync_copy(qn_v, q_out)

        plsc.subcore_barrier()
        wid = sid * 2 + cid

        @pl.when(wid < n_workers)
        def _():
            base = wid * c
            pltpu.sync_copy(perm_sh.at[pl.ds(base, c)], idx_v)
            for tb, buf, out in zip(tabs, bufs, outs):
                pltpu.async_copy(tb.at[idx_v], buf, sem).wait()
                pltpu.sync_copy(buf, out.at[pl.ds(base, c)])

    fn = pl.kernel(body, out_type=tuple(out_type), mesh=mesh,
                   scratch_types=tuple(scratch),
                   compiler_params=pltpu.CompilerParams(
                       needs_layout_passes=False))
    if with_q:
        return fn(rank_flat, s_flat, q_flat, *tables)
    return fn(rank_flat, s_flat, *tables)


def _readout_body(x_ref, v_ref, mx_ref, sm_ref):
    i = pl.program_id(0)
    xs = x_ref[...] * jnp.tanh(v_ref[...])
    m = jnp.max(xs, axis=0, keepdims=True)
    s = jnp.sum(xs, axis=0, keepdims=True)

    @pl.when(i == 0)
    def _():
        mx_ref[...] = m
        sm_ref[...] = s

    @pl.when(i > 0)
    def _():
        mx_ref[...] = jnp.maximum(mx_ref[...], m)
        sm_ref[...] += s


def _readout(x_gath, vals_col, br):
    n, nhid = x_gath.shape
    return pl.pallas_call(
        _readout_body,
        grid=(n // br,),
        in_specs=[
            pl.BlockSpec((br, nhid), lambda i: (i, 0)),
            pl.BlockSpec((br, 1), lambda i: (i, 0)),
        ],
        out_specs=[
            pl.BlockSpec((1, nhid), lambda i: (0, 0)),
            pl.BlockSpec((1, nhid), lambda i: (0, 0)),
        ],
        out_shape=[
            jax.ShapeDtypeStruct((1, nhid), F32),
            jax.ShapeDtypeStruct((1, nhid), F32),
        ],
    )(x_gath, vals_col)


def kernel(feature, img_info, W_pos, b_pos, g_f, b_f, g_p, b_p,
           W1, b1, W2, b2, W3, b3, Ws1, bs1, Ws2, bs2, Ws3, bs3):
    n = feature.shape[0]
    nf = feature.shape[1]
    nhid = W1.shape[1]
    k1 = math.ceil(0.75 * n)
    k2 = math.ceil(0.75 * k1)
    k3 = math.ceil(0.75 * k2)

    img_p = jnp.pad(img_info, ((0, 0), (0, 128 - img_info.shape[1])))
    wp_p = jnp.pad(W_pos, ((0, 128 - W_pos.shape[0]), (0, 128 - W_pos.shape[1])))
    bp_p = jnp.pad(b_pos, (0, 128 - b_pos.shape[0])).reshape(1, 128)
    feat_p = jnp.pad(feature, ((0, 0), (0, 512 - nf)))
    gf_p = jnp.pad(g_f, (0, 512 - nf)).reshape(1, 512)
    bf_p = jnp.pad(b_f, (0, 512 - nf)).reshape(1, 512)
    gp_p = jnp.pad(g_p, (0, 128 - g_p.shape[0])).reshape(1, 128)
    bpl = jnp.pad(b_p, (0, 128 - b_p.shape[0])).reshape(1, 128)
    w1p = jnp.pad(W1[nf:, :], ((0, 128 - (512 - nf)), (0, 0)))
    ws1_p = jnp.pad(Ws1, ((0, 0), (0, 127)))
    ws2_p = jnp.pad(Ws2, ((0, 0), (0, 127)))
    ws3_p = jnp.pad(Ws3, ((0, 0), (0, 127)))
    bs1_r = jnp.broadcast_to(bs1.reshape(1, 1), (1, 128))
    bs2_r = jnp.broadcast_to(bs2.reshape(1, 1), (1, 128))
    bs3_r = jnp.broadcast_to(bs3.reshape(1, 1), (1, 128))
    b1_r = b1.reshape(1, nhid)
    b2_r = b2.reshape(1, nhid)
    b3_r = b3.reshape(1, nhid)

    lnf, lnp, xw1 = _attn_ln(img_p, wp_p, bp_p, feat_p, gf_p, bf_p, gp_p,
                             bpl, W1, w1p, nhid)
    feat = jnp.concatenate([lnf[:, :nf], lnp[:, :512 - nf]], axis=1)

    d_bf, colmax = _dist_max(feat, 512, 2048)
    t = colmax

    mask1, dinv1 = _mask1_deg(d_bf, t, 1024, 2048)
    x1 = _conv(mask1, xw1, dinv1, b1_r, None, None, 1024, 4096)
    s1 = _score(mask1, x1, dinv1, ws1_p, bs1_r, 1024, 4096)
    rank1 = _rank(s1, s1.reshape(1, n))
    perm1, vals1, featsub2, xg1 = _select_gather(
        rank1.reshape(n), s1.reshape(n), None, k1, [feat, x1])

    q2c = perm1.reshape(k1, 1)
    q2r = perm1.reshape(1, k1)
    mask2, dinv2 = _mask23_deg(featsub2, q2c, q2r, t, 1024, 3072)
    x2, mx1, sm1 = _conv(mask2, xg1, dinv2, b2_r, vals1.reshape(k1, 1), W2,
                         1024, 3072)
    read1 = jnp.concatenate([mx1, sm1 / k1], axis=1)
    s2 = _score(mask2, x2, dinv2, ws2_p, bs2_r, 1024, 3072)
    rank2 = _rank(s2, s2.reshape(1, k1))
    perm2, vals2, q3, featsub3, xg2 = _select_gather(
        rank2.reshape(k1), s2.reshape(k1), perm1, k2, [featsub2, x2])

    q3c = q3.reshape(k2, 1)
    q3r = q3.reshape(1, k2)
    mask3, dinv3 = _mask23_deg(featsub3, q3c, q3r, t, 768, 2304)
    x3, mx2, sm2 = _conv(mask3, xg2, dinv3, b3_r, vals2.reshape(k2, 1), W3,
                         768, 2304)
    read2 = jnp.concatenate([mx2, sm2 / k2], axis=1)
    s3 = _score(mask3, x3, dinv3, ws3_p, bs3_r, 768, 2304)
    rank3 = _rank(s3, s3.reshape(1, k2))
    perm3, vals3, xg3 = _select_gather(
        rank3.reshape(k2), s3.reshape(k2), None, k3, [x3])
    mx3, sm3 = _readout(xg3, vals3.reshape(k3, 1), 192)
    read3 = jnp.concatenate([mx3, sm3 / k3], axis=1)

    return read1 + read2 + read3
